# Initial kernel scaffold; baseline (speedup 1.0000x reference)
import functools

import jax
import jax.numpy as jnp
from jax import lax
from jax.experimental import pallas as pl
from jax.experimental.pallas import tpu as pltpu
from jax.experimental.pallas import tpu_sc as plsc

N = 10000
E = 320000
D = 128

NPAD = 10240
EPAD = 327680
NC = 2
NS = 16
GPT = EPAD // (NC * NS * 128)
EDGE_ROWS = EPAD // 128
RPT = NPAD // NS


def _sc_edge_body(h_hbm, asrc_hbm, adst_hbm, src_hbm, dst_hbm,
                  outp_hbm, dnp_hbm,
                  asrc_v, adst_v, src_v, dst_v, w_v, rows_v, zrow_v,
                  shared_out, shared_dn):
    c = lax.axis_index("c")
    s = lax.axis_index("s")
    wid = c * NS + s

    pltpu.sync_copy(asrc_hbm, asrc_v)
    pltpu.sync_copy(adst_hbm, adst_v)
    pltpu.sync_copy(src_hbm.at[pl.ds(wid * GPT, GPT)], src_v)
    pltpu.sync_copy(dst_hbm.at[pl.ds(wid * GPT, GPT)], dst_v)

    zero16 = jnp.zeros((16,), jnp.float32)

    @pl.loop(0, 128)
    def _zero_rows(r):
        for k in range(8):
            rows_v[r, pl.ds(k * 16, 16)] = zero16

    @pl.loop(0, RPT // 16)
    def _zero_zrow(i):
        zrow_v[pl.ds(i * 16, 16)] = zero16

    for t in range(RPT // 128):
        pltpu.sync_copy(rows_v, shared_out.at[pl.ds(s * RPT + t * 128, 128)])
    pltpu.sync_copy(zrow_v, shared_dn.at[pl.ds(s * RPT, RPT)])
    plsc.subcore_barrier()

    @pl.loop(0, GPT)
    def _phase_a(j):
        for k in range(8):
            sv = src_v[j, pl.ds(k * 16, 16)]
            dv = dst_v[j, pl.ds(k * 16, 16)]
            av = plsc.load_gather(asrc_v, [sv])
            bv = plsc.load_gather(adst_v, [dv])
            e = av + bv
            e = jnp.where(e > 0.0, e, 0.2 * e)
            w_v[j, pl.ds(k * 16, 16)] = jnp.exp(e)
        pltpu.sync_copy(w_v.at[j], shared_dn.at[dst_v.at[j]], add=True)

    @pl.loop(0, GPT)
    def _phase_b(j):
        pltpu.sync_copy(h_hbm.at[src_v.at[j]], rows_v)

        @pl.loop(0, 128)
        def _scale(r):
            wb = lax.broadcast(w_v[j, r], (16,))
            for k in range(8):
                rows_v[r, pl.ds(k * 16, 16)] = rows_v[r, pl.ds(k * 16, 16)] * wb

        pltpu.sync_copy(rows_v, shared_out.at[dst_v.at[j]], add=True)

    plsc.subcore_barrier()

    for t in range(RPT // 128):
        pltpu.sync_copy(shared_out.at[pl.ds(s * RPT + t * 128, 128)],
                        outp_hbm.at[c, pl.ds(s * RPT + t * 128, 128)])
    pltpu.sync_copy(shared_dn.at[pl.ds(s * RPT, RPT)],
                    dnp_hbm.at[c, pl.ds(s * RPT, RPT)])


def _build_sc_edge(interpret=False):
    mesh = plsc.VectorSubcoreMesh(core_axis_name="c", subcore_axis_name="s",
                                  num_cores=NC, num_subcores=NS)
    return pl.kernel(
        _sc_edge_body,
        out_type=[
            jax.ShapeDtypeStruct((NC, NPAD, D), jnp.float32),
            jax.ShapeDtypeStruct((NC, NPAD), jnp.float32),
        ],
        mesh=mesh,
        scratch_types=[
            pltpu.VMEM((NPAD,), jnp.float32),
            pltpu.VMEM((NPAD,), jnp.float32),
            pltpu.VMEM((GPT, 128), jnp.int32),
            pltpu.VMEM((GPT, 128), jnp.int32),
            pltpu.VMEM((GPT, 128), jnp.float32),
            pltpu.VMEM((128, D), jnp.float32),
            pltpu.VMEM((RPT,), jnp.float32),
            pltpu.VMEM_SHARED((NPAD, D), jnp.float32),
            pltpu.VMEM_SHARED((NPAD,), jnp.float32),
        ],
        interpret=interpret,
    )


_BM = 256
_GRID = NPAD // _BM


def _mm_kernel(x_ref, w_ref, a_ref, h_ref, av_ref):
    xb = x_ref[...]
    h_ref[...] = jnp.dot(xb, w_ref[...], preferred_element_type=jnp.float32)
    av_ref[...] = jnp.dot(xb, a_ref[...], preferred_element_type=jnp.float32)


def _build_tc_matmul(interpret=False):
    return pl.pallas_call(
        _mm_kernel,
        grid=(_GRID,),
        in_specs=[
            pl.BlockSpec((_BM, D), lambda i: (i, 0)),
            pl.BlockSpec((D, D), lambda i: (0, 0)),
            pl.BlockSpec((D, 128), lambda i: (0, 0)),
        ],
        out_specs=[
            pl.BlockSpec((_BM, D), lambda i: (i, 0)),
            pl.BlockSpec((_BM, 128), lambda i: (i, 0)),
        ],
        out_shape=[
            jax.ShapeDtypeStruct((NPAD, D), jnp.float32),
            jax.ShapeDtypeStruct((NPAD, 128), jnp.float32),
        ],
        interpret=interpret,
    )


def _epi_mm_kernel(p_ref, dn_ref, b_ref, w_ref, a_ref, h_ref, av_ref):
    p = p_ref[0] + p_ref[1]
    dn = dn_ref[0] + dn_ref[1] + 1e-16
    y = p / dn + b_ref[...]
    y = jnp.maximum(y, 0.0)
    h_ref[...] = jnp.dot(y, w_ref[...], preferred_element_type=jnp.float32)
    av_ref[...] = jnp.dot(y, a_ref[...], preferred_element_type=jnp.float32)


def _build_tc_epi_matmul(interpret=False):
    return pl.pallas_call(
        _epi_mm_kernel,
        grid=(_GRID,),
        in_specs=[
            pl.BlockSpec((NC, _BM, D), lambda i: (0, i, 0)),
            pl.BlockSpec((NC, _BM, 1), lambda i: (0, i, 0)),
            pl.BlockSpec((1, D), lambda i: (0, 0)),
            pl.BlockSpec((D, D), lambda i: (0, 0)),
            pl.BlockSpec((D, 128), lambda i: (0, 0)),
        ],
        out_specs=[
            pl.BlockSpec((_BM, D), lambda i: (i, 0)),
            pl.BlockSpec((_BM, 128), lambda i: (i, 0)),
        ],
        out_shape=[
            jax.ShapeDtypeStruct((NPAD, D), jnp.float32),
            jax.ShapeDtypeStruct((NPAD, 128), jnp.float32),
        ],
        interpret=interpret,
    )


def _epi_kernel(p_ref, dn_ref, b_ref, o_ref):
    p = p_ref[0] + p_ref[1]
    dn = dn_ref[0] + dn_ref[1] + 1e-16
    o_ref[...] = p / dn + b_ref[...]


def _build_tc_epilogue(interpret=False):
    return pl.pallas_call(
        _epi_kernel,
        grid=(_GRID,),
        in_specs=[
            pl.BlockSpec((NC, _BM, D), lambda i: (0, i, 0)),
            pl.BlockSpec((NC, _BM, 1), lambda i: (0, i, 0)),
            pl.BlockSpec((1, D), lambda i: (0, 0)),
        ],
        out_specs=pl.BlockSpec((_BM, D), lambda i: (i, 0)),
        out_shape=jax.ShapeDtypeStruct((NPAD, D), jnp.float32),
        interpret=interpret,
    )


_sc_edge = _build_sc_edge()
_tc_matmul = _build_tc_matmul()
_tc_epi_matmul = _build_tc_epi_matmul()
_tc_epilogue = _build_tc_epilogue()


@jax.jit
def kernel(x, edge_index, W1, a1_src, a1_dst, b1, W2, a2_src, a2_dst, b2):
    src = edge_index[0].astype(jnp.int32)
    dst = edge_index[1].astype(jnp.int32)
    pad_idx = N + (jnp.arange(EPAD - E, dtype=jnp.int32) % (NPAD - N))
    src2d = jnp.concatenate([src, pad_idx]).reshape(EDGE_ROWS, 128)
    dst2d = jnp.concatenate([dst, pad_idx]).reshape(EDGE_ROWS, 128)

    xp = jnp.zeros((NPAD, D), jnp.float32).at[:N].set(x)
    A1 = (jnp.zeros((D, 128), jnp.float32)
          .at[:, 0].set(a1_src).at[:, 1].set(a1_dst))
    A2 = (jnp.zeros((D, 128), jnp.float32)
          .at[:, 0].set(a2_src).at[:, 1].set(a2_dst))

    h1, av1 = _tc_matmul(xp, W1, A1)
    outp1, dnp1 = _sc_edge(h1, av1[:, 0], av1[:, 1], src2d, dst2d)
    h2, av2 = _tc_epi_matmul(outp1, dnp1.reshape(NC, NPAD, 1),
                             b1.reshape(1, D), W2, A2)
    outp2, dnp2 = _sc_edge(h2, av2[:, 0], av2[:, 1], src2d, dst2d)
    out = _tc_epilogue(outp2, dnp2.reshape(NC, NPAD, 1), b2.reshape(1, D))
    return out[:N]


# speedup vs baseline: 25.5240x; 25.5240x over previous
import functools

import jax
import jax.numpy as jnp
from jax import lax
from jax.experimental import pallas as pl
from jax.experimental.pallas import tpu as pltpu
from jax.experimental.pallas import tpu_sc as plsc

N = 10000
E = 320000
D = 128

NPAD = 10240
EPAD = 327680
NC = 2
NS = 16
GPT = EPAD // (NC * NS * 128)
EDGE_ROWS = EPAD // 128
RPT = NPAD // NS


def _sc_edge_body(h_hbm, asrc_hbm, adst_hbm, src_hbm, dst_hbm,
                  outp_hbm, dnp_hbm,
                  src_v, dst_v, av_v, bv_v, w_v, rows_v,
                  shared_out, shared_dn, shared_asrc, shared_adst):
    c = lax.axis_index("c")
    s = lax.axis_index("s")
    wid = c * NS + s

    @pl.when(s == 0)
    def _stage_alpha():
        pltpu.sync_copy(asrc_hbm, shared_asrc)
        pltpu.sync_copy(adst_hbm, shared_adst)

    zero16 = jnp.zeros((16,), jnp.float32)

    @pl.loop(0, 128)
    def _zero_rows(r):
        for m in range(8):
            rows_v[0, r, pl.ds(m * 16, 16)] = zero16
        for k in range(8):
            w_v[0, pl.ds(k * 16, 16)] = zero16

    for t in range(RPT // 128):
        pltpu.sync_copy(rows_v.at[0],
                        shared_out.at[pl.ds(s * RPT + t * 128, 128)])
        pltpu.sync_copy(w_v.at[0],
                        shared_dn.at[pl.ds(s * RPT + t * 128, 128)])
    plsc.subcore_barrier()

    @pl.loop(0, GPT)
    def _edges(j):
        g = wid * GPT + j
        b = 0
        pltpu.sync_copy(src_hbm.at[pl.ds(g, 1)], src_v.at[pl.ds(b, 1)])
        pltpu.sync_copy(dst_hbm.at[pl.ds(g, 1)], dst_v.at[pl.ds(b, 1)])
        pltpu.sync_copy(shared_asrc.at[src_v.at[b]], av_v.at[b])
        pltpu.sync_copy(shared_adst.at[dst_v.at[b]], bv_v.at[b])
        for k in range(8):
            av = av_v[b, pl.ds(k * 16, 16)]
            bv = bv_v[b, pl.ds(k * 16, 16)]
            e = av + bv
            e = jnp.where(e > 0.0, e, 0.2 * e)
            w_v[b, pl.ds(k * 16, 16)] = jnp.exp(e)
        pltpu.sync_copy(w_v.at[b], shared_dn.at[dst_v.at[b]], add=True)
        pltpu.sync_copy(h_hbm.at[src_v.at[b]], rows_v.at[b])
        for k in range(8):
            wvec = w_v[b, pl.ds(k * 16, 16)]
            for l in range(16):
                wb = lax.broadcast(wvec[l], (16,))
                r = k * 16 + l
                for m in range(8):
                    rows_v[b, r, pl.ds(m * 16, 16)] = (
                        rows_v[b, r, pl.ds(m * 16, 16)] * wb)
        pltpu.sync_copy(rows_v.at[b], shared_out.at[dst_v.at[b]], add=True)

    plsc.subcore_barrier()

    for t in range(RPT // 128):
        pltpu.sync_copy(shared_out.at[pl.ds(s * RPT + t * 128, 128)],
                        outp_hbm.at[c, pl.ds(s * RPT + t * 128, 128)])
    pltpu.sync_copy(shared_dn.at[pl.ds(s * RPT, RPT)],
                    dnp_hbm.at[c, pl.ds(s * RPT, RPT)])


def _build_sc_edge(interpret=False):
    mesh = plsc.VectorSubcoreMesh(core_axis_name="c", subcore_axis_name="s",
                                  num_cores=NC, num_subcores=NS)
    return pl.kernel(
        _sc_edge_body,
        out_type=[
            jax.ShapeDtypeStruct((NC, NPAD, D), jnp.float32),
            jax.ShapeDtypeStruct((NC, NPAD), jnp.float32),
        ],
        mesh=mesh,
        scratch_types=[
            pltpu.VMEM((2, 128), jnp.int32),
            pltpu.VMEM((2, 128), jnp.int32),
            pltpu.VMEM((2, 128), jnp.float32),
            pltpu.VMEM((2, 128), jnp.float32),
            pltpu.VMEM((2, 128), jnp.float32),
            pltpu.VMEM((2, 128, D), jnp.float32),
            pltpu.VMEM_SHARED((NPAD, D), jnp.float32),
            pltpu.VMEM_SHARED((NPAD,), jnp.float32),
            pltpu.VMEM_SHARED((NPAD,), jnp.float32),
            pltpu.VMEM_SHARED((NPAD,), jnp.float32),
        ],
        compiler_params=pltpu.CompilerParams(needs_layout_passes=False),
        interpret=interpret,
    )


_BM = 256
_GRID = NPAD // _BM


def _mm_kernel(x_ref, w_ref, a_ref, h_ref, av_ref):
    xb = x_ref[...]
    h = jnp.dot(xb, w_ref[...], preferred_element_type=jnp.float32)
    h_ref[...] = h
    av_ref[...] = jnp.dot(h, a_ref[...], preferred_element_type=jnp.float32)


def _build_tc_matmul(interpret=False):
    return pl.pallas_call(
        _mm_kernel,
        grid=(_GRID,),
        in_specs=[
            pl.BlockSpec((_BM, D), lambda i: (i, 0)),
            pl.BlockSpec((D, D), lambda i: (0, 0)),
            pl.BlockSpec((D, 128), lambda i: (0, 0)),
        ],
        out_specs=[
            pl.BlockSpec((_BM, D), lambda i: (i, 0)),
            pl.BlockSpec((_BM, 128), lambda i: (i, 0)),
        ],
        out_shape=[
            jax.ShapeDtypeStruct((NPAD, D), jnp.float32),
            jax.ShapeDtypeStruct((NPAD, 128), jnp.float32),
        ],
        interpret=interpret,
    )


def _epi_mm_kernel(p_ref, dn_ref, b_ref, w_ref, a_ref, h_ref, av_ref):
    p = p_ref[0] + p_ref[1]
    dn = dn_ref[0] + dn_ref[1] + 1e-16
    y = p / dn + b_ref[...]
    y = jnp.maximum(y, 0.0)
    h = jnp.dot(y, w_ref[...], preferred_element_type=jnp.float32)
    h_ref[...] = h
    av_ref[...] = jnp.dot(h, a_ref[...], preferred_element_type=jnp.float32)


def _build_tc_epi_matmul(interpret=False):
    return pl.pallas_call(
        _epi_mm_kernel,
        grid=(_GRID,),
        in_specs=[
            pl.BlockSpec((NC, _BM, D), lambda i: (0, i, 0)),
            pl.BlockSpec((NC, _BM, 1), lambda i: (0, i, 0)),
            pl.BlockSpec((1, D), lambda i: (0, 0)),
            pl.BlockSpec((D, D), lambda i: (0, 0)),
            pl.BlockSpec((D, 128), lambda i: (0, 0)),
        ],
        out_specs=[
            pl.BlockSpec((_BM, D), lambda i: (i, 0)),
            pl.BlockSpec((_BM, 128), lambda i: (i, 0)),
        ],
        out_shape=[
            jax.ShapeDtypeStruct((NPAD, D), jnp.float32),
            jax.ShapeDtypeStruct((NPAD, 128), jnp.float32),
        ],
        interpret=interpret,
    )


def _epi_kernel(p_ref, dn_ref, b_ref, o_ref):
    p = p_ref[0] + p_ref[1]
    dn = dn_ref[0] + dn_ref[1] + 1e-16
    o_ref[...] = p / dn + b_ref[...]


def _build_tc_epilogue(interpret=False):
    return pl.pallas_call(
        _epi_kernel,
        grid=(_GRID,),
        in_specs=[
            pl.BlockSpec((NC, _BM, D), lambda i: (0, i, 0)),
            pl.BlockSpec((NC, _BM, 1), lambda i: (0, i, 0)),
            pl.BlockSpec((1, D), lambda i: (0, 0)),
        ],
        out_specs=pl.BlockSpec((_BM, D), lambda i: (i, 0)),
        out_shape=jax.ShapeDtypeStruct((NPAD, D), jnp.float32),
        interpret=interpret,
    )


_build_sc_edge = functools.lru_cache(maxsize=None)(_build_sc_edge)
_build_tc_matmul = functools.lru_cache(maxsize=None)(_build_tc_matmul)
_build_tc_epi_matmul = functools.lru_cache(maxsize=None)(_build_tc_epi_matmul)
_build_tc_epilogue = functools.lru_cache(maxsize=None)(_build_tc_epilogue)


@jax.jit
def kernel(x, edge_index, W1, a1_src, a1_dst, b1, W2, a2_src, a2_dst, b2):
    src = edge_index[0].astype(jnp.int32)
    dst = edge_index[1].astype(jnp.int32)
    pad_idx = N + (jnp.arange(EPAD - E, dtype=jnp.int32) % (NPAD - N))
    src2d = jnp.concatenate([src, pad_idx]).reshape(EDGE_ROWS, 128)
    dst2d = jnp.concatenate([dst, pad_idx]).reshape(EDGE_ROWS, 128)

    xp = jnp.zeros((NPAD, D), jnp.float32).at[:N].set(x)
    A1 = (jnp.zeros((D, 128), jnp.float32)
          .at[:, 0].set(a1_src).at[:, 1].set(a1_dst))
    A2 = (jnp.zeros((D, 128), jnp.float32)
          .at[:, 0].set(a2_src).at[:, 1].set(a2_dst))

    sc_edge = _build_sc_edge()
    h1, av1 = _build_tc_matmul()(xp, W1, A1)
    outp1, dnp1 = sc_edge(h1, av1[:, 0], av1[:, 1], src2d, dst2d)
    h2, av2 = _build_tc_epi_matmul()(outp1, dnp1.reshape(NC, NPAD, 1),
                                     b1.reshape(1, D), W2, A2)
    outp2, dnp2 = sc_edge(h2, av2[:, 0], av2[:, 1], src2d, dst2d)
    out = _build_tc_epilogue()(outp2, dnp2.reshape(NC, NPAD, 1),
                               b2.reshape(1, D))
    return out[:N]


# speedup vs baseline: 36.8084x; 1.4421x over previous
import functools

import jax
import jax.numpy as jnp
from jax import lax
from jax.experimental import pallas as pl
from jax.experimental.pallas import tpu as pltpu
from jax.experimental.pallas import tpu_sc as plsc

N = 10000
E = 320000
D = 128

NPAD = 10240
EPAD = 327680
NC = 2
NS = 16
GPT = EPAD // (NC * NS * 128)
EDGE_ROWS = EPAD // 128
RPT = NPAD // NS


def _sc_edge_body(h_hbm, asrc_hbm, adst_hbm, edges_hbm,
                  outp_hbm, dnp_hbm,
                  e_v, av_v, bv_v, w_v, rows_v,
                  shared_out, shared_dn, shared_asrc, shared_adst,
                  sem_g0, sem_g1, sem_s0, sem_s1):
    c = lax.axis_index("c")
    s = lax.axis_index("s")
    wid = c * NS + s
    sem_g = (sem_g0, sem_g1)
    sem_s = (sem_s0, sem_s1)

    @pl.when(s == 0)
    def _stage_alpha():
        pltpu.sync_copy(asrc_hbm, shared_asrc)
        pltpu.sync_copy(adst_hbm, shared_adst)

    zero16 = jnp.zeros((16,), jnp.float32)

    @pl.loop(0, 128)
    def _zero_rows(r):
        for m in range(8):
            rows_v[0, r, pl.ds(m * 16, 16)] = zero16
        for k in range(8):
            w_v[0, pl.ds(k * 16, 16)] = zero16

    for t in range(RPT // 128):
        pltpu.sync_copy(rows_v.at[0],
                        shared_out.at[pl.ds(s * RPT + t * 128, 128)])
        pltpu.sync_copy(w_v.at[0],
                        shared_dn.at[pl.ds(s * RPT + t * 128, 128)])
    plsc.subcore_barrier()


    def prep(jn, slot, pb):
        pltpu.sync_copy(edges_hbm.at[pl.ds(wid * GPT + jn, 1)],
                        e_v.at[pl.ds(slot, 1)])
        pltpu.sync_copy(shared_asrc.at[e_v.at[slot, 0]], av_v)
        pltpu.sync_copy(shared_adst.at[e_v.at[slot, 1]], bv_v)
        for k in range(8):
            e = av_v[pl.ds(k * 16, 16)] + bv_v[pl.ds(k * 16, 16)]
            e = jnp.where(e > 0.0, e, 0.2 * e)
            w_v[pb, pl.ds(k * 16, 16)] = jnp.exp(e)

    def dn_scatter(slot, pb):
        pltpu.sync_copy(w_v.at[pb], shared_dn.at[e_v.at[slot, 1]], add=True)

    def start_gather(slot, rb):
        pltpu.async_copy(h_hbm.at[e_v.at[slot, 0]], rows_v.at[rb], sem_g[rb])

    def wait_gather(slot, rb):
        pltpu.make_async_copy(h_hbm.at[e_v.at[slot, 0]], rows_v.at[rb],
                              sem_g[rb]).wait()

    def start_scatter(slot, rb):
        pltpu.async_copy(rows_v.at[rb], shared_out.at[e_v.at[slot, 1]],
                         sem_s[rb], add=True)

    def wait_scatter(slot, rb):
        pltpu.make_async_copy(rows_v.at[rb], shared_out.at[e_v.at[slot, 1]],
                              sem_s[rb]).wait()

    def scale(rb, pb):
        @pl.loop(0, 8)
        def _scale(k):
            wvec = w_v[pb, pl.ds(k * 16, 16)]
            for l in range(16):
                wb = lax.broadcast(wvec[l], (16,))
                r = k * 16 + l
                for m in range(8):
                    rows_v[rb, r, pl.ds(m * 16, 16)] = (
                        rows_v[rb, r, pl.ds(m * 16, 16)] * wb)

    def body(j, b, sl, sln, slp, first=False):
        wait_gather(sl, b)
        dn_scatter(sl, b)
        prep(j + 1, sln, 1 - b)
        if not first:
            wait_scatter(slp, 1 - b)
        start_gather(sln, 1 - b)
        scale(b, b)
        start_scatter(sl, b)

    prep(0, 0, 0)
    start_gather(0, 0)
    body(0, 0, 0, 1, 3, first=True)
    body(1, 1, 1, 2, 0)
    body(2, 0, 2, 3, 1)
    body(3, 1, 3, 0, 2)

    @pl.loop(0, (GPT - 4) // 4)
    def _edges(q):
        j = 4 + q * 4
        body(j, 0, 0, 1, 3)
        body(j + 1, 1, 1, 2, 0)
        body(j + 2, 0, 2, 3, 1)
        body(j + 3, 1, 3, 0, 2)

    wait_gather(0, 0)
    wait_scatter(3, 1)

    plsc.subcore_barrier()

    for t in range(RPT // 128):
        pltpu.sync_copy(shared_out.at[pl.ds(s * RPT + t * 128, 128)],
                        outp_hbm.at[c, pl.ds(s * RPT + t * 128, 128)])
    pltpu.sync_copy(shared_dn.at[pl.ds(s * RPT, RPT)],
                    dnp_hbm.at[c, pl.ds(s * RPT, RPT)])


def _build_sc_edge(interpret=False):
    mesh = plsc.VectorSubcoreMesh(core_axis_name="c", subcore_axis_name="s",
                                  num_cores=NC, num_subcores=NS)
    return pl.kernel(
        _sc_edge_body,
        out_type=[
            jax.ShapeDtypeStruct((NC, NPAD, D), jnp.float32),
            jax.ShapeDtypeStruct((NC, NPAD), jnp.float32),
        ],
        mesh=mesh,
        scratch_types=[
            pltpu.VMEM((4, 2, 128), jnp.int32),
            pltpu.VMEM((128,), jnp.float32),
            pltpu.VMEM((128,), jnp.float32),
            pltpu.VMEM((2, 128), jnp.float32),
            pltpu.VMEM((2, 128, D), jnp.float32),
            pltpu.VMEM_SHARED((NPAD, D), jnp.float32),
            pltpu.VMEM_SHARED((NPAD,), jnp.float32),
            pltpu.VMEM_SHARED((NPAD,), jnp.float32),
            pltpu.VMEM_SHARED((NPAD,), jnp.float32),
            pltpu.SemaphoreType.DMA,
            pltpu.SemaphoreType.DMA,
            pltpu.SemaphoreType.DMA,
            pltpu.SemaphoreType.DMA,
        ],
        compiler_params=pltpu.CompilerParams(needs_layout_passes=False),
        interpret=interpret,
    )


_BM = 256
_GRID = NPAD // _BM


def _mm_kernel(x_ref, w_ref, a_ref, h_ref, av_ref):
    xb = x_ref[...]
    h = jnp.dot(xb, w_ref[...], preferred_element_type=jnp.float32)
    h_ref[...] = h
    av_ref[...] = jnp.dot(h, a_ref[...], preferred_element_type=jnp.float32)


def _build_tc_matmul(interpret=False):
    return pl.pallas_call(
        _mm_kernel,
        grid=(_GRID,),
        in_specs=[
            pl.BlockSpec((_BM, D), lambda i: (i, 0)),
            pl.BlockSpec((D, D), lambda i: (0, 0)),
            pl.BlockSpec((D, 128), lambda i: (0, 0)),
        ],
        out_specs=[
            pl.BlockSpec((_BM, D), lambda i: (i, 0)),
            pl.BlockSpec((_BM, 128), lambda i: (i, 0)),
        ],
        out_shape=[
            jax.ShapeDtypeStruct((NPAD, D), jnp.float32),
            jax.ShapeDtypeStruct((NPAD, 128), jnp.float32),
        ],
        interpret=interpret,
    )


def _epi_mm_kernel(p_ref, dn_ref, b_ref, w_ref, a_ref, h_ref, av_ref):
    p = p_ref[0] + p_ref[1]
    dn = dn_ref[0] + dn_ref[1] + 1e-16
    y = p / dn + b_ref[...]
    y = jnp.maximum(y, 0.0)
    h = jnp.dot(y, w_ref[...], preferred_element_type=jnp.float32)
    h_ref[...] = h
    av_ref[...] = jnp.dot(h, a_ref[...], preferred_element_type=jnp.float32)


def _build_tc_epi_matmul(interpret=False):
    return pl.pallas_call(
        _epi_mm_kernel,
        grid=(_GRID,),
        in_specs=[
            pl.BlockSpec((NC, _BM, D), lambda i: (0, i, 0)),
            pl.BlockSpec((NC, _BM, 1), lambda i: (0, i, 0)),
            pl.BlockSpec((1, D), lambda i: (0, 0)),
            pl.BlockSpec((D, D), lambda i: (0, 0)),
            pl.BlockSpec((D, 128), lambda i: (0, 0)),
        ],
        out_specs=[
            pl.BlockSpec((_BM, D), lambda i: (i, 0)),
            pl.BlockSpec((_BM, 128), lambda i: (i, 0)),
        ],
        out_shape=[
            jax.ShapeDtypeStruct((NPAD, D), jnp.float32),
            jax.ShapeDtypeStruct((NPAD, 128), jnp.float32),
        ],
        interpret=interpret,
    )


def _epi_kernel(p_ref, dn_ref, b_ref, o_ref):
    p = p_ref[0] + p_ref[1]
    dn = dn_ref[0] + dn_ref[1] + 1e-16
    o_ref[...] = p / dn + b_ref[...]


def _build_tc_epilogue(interpret=False):
    return pl.pallas_call(
        _epi_kernel,
        grid=(_GRID,),
        in_specs=[
            pl.BlockSpec((NC, _BM, D), lambda i: (0, i, 0)),
            pl.BlockSpec((NC, _BM, 1), lambda i: (0, i, 0)),
            pl.BlockSpec((1, D), lambda i: (0, 0)),
        ],
        out_specs=pl.BlockSpec((_BM, D), lambda i: (i, 0)),
        out_shape=jax.ShapeDtypeStruct((NPAD, D), jnp.float32),
        interpret=interpret,
    )


_build_sc_edge = functools.lru_cache(maxsize=None)(_build_sc_edge)
_build_tc_matmul = functools.lru_cache(maxsize=None)(_build_tc_matmul)
_build_tc_epi_matmul = functools.lru_cache(maxsize=None)(_build_tc_epi_matmul)
_build_tc_epilogue = functools.lru_cache(maxsize=None)(_build_tc_epilogue)


@jax.jit
def kernel(x, edge_index, W1, a1_src, a1_dst, b1, W2, a2_src, a2_dst, b2):
    src = edge_index[0].astype(jnp.int32)
    dst = edge_index[1].astype(jnp.int32)
    pad_idx = N + (jnp.arange(EPAD - E, dtype=jnp.int32) % (NPAD - N))
    src2d = jnp.concatenate([src, pad_idx]).reshape(EDGE_ROWS, 128)
    dst2d = jnp.concatenate([dst, pad_idx]).reshape(EDGE_ROWS, 128)
    e3 = jnp.stack([src2d, dst2d], axis=1)
    extra = jnp.broadcast_to(N + (jnp.arange(128, dtype=jnp.int32)
                                  % (NPAD - N)), (8, 2, 128))
    e3 = jnp.concatenate([e3, extra], axis=0)

    xp = jnp.zeros((NPAD, D), jnp.float32).at[:N].set(x)
    A1 = (jnp.zeros((D, 128), jnp.float32)
          .at[:, 0].set(a1_src).at[:, 1].set(a1_dst))
    A2 = (jnp.zeros((D, 128), jnp.float32)
          .at[:, 0].set(a2_src).at[:, 1].set(a2_dst))

    sc_edge = _build_sc_edge()
    h1, av1 = _build_tc_matmul()(xp, W1, A1)
    outp1, dnp1 = sc_edge(h1, av1[:, 0], av1[:, 1], e3)
    h2, av2 = _build_tc_epi_matmul()(outp1, dnp1.reshape(NC, NPAD, 1),
                                     b1.reshape(1, D), W2, A2)
    outp2, dnp2 = sc_edge(h2, av2[:, 0], av2[:, 1], e3)
    out = _build_tc_epilogue()(outp2, dnp2.reshape(NC, NPAD, 1),
                               b2.reshape(1, D))
    return out[:N]


# speedup vs baseline: 41.3246x; 1.1227x over previous
import functools

import jax
import jax.numpy as jnp
from jax import lax
from jax.experimental import pallas as pl
from jax.experimental.pallas import tpu as pltpu
from jax.experimental.pallas import tpu_sc as plsc

N = 10000
E = 320000
D = 128

NPAD = 10240
EPAD = 327680
NC = 2
NS = 16
GPT = EPAD // (NC * NS * 128)
EDGE_ROWS = EPAD // 128
RPT = NPAD // NS


def _sc_edge_body(h_hbm, asrc_hbm, adst_hbm, edges_hbm,
                  outp_hbm, dnp_hbm,
                  e_v, av_v, bv_v, w_v, rows_v,
                  shared_out, shared_dn, shared_asrc, shared_adst,
                  sem_g0, sem_g1, sem_s0, sem_s1):
    c = lax.axis_index("c")
    s = lax.axis_index("s")
    wid = c * NS + s
    sem_g = (sem_g0, sem_g1)
    sem_s = (sem_s0, sem_s1)

    @pl.when(s == 0)
    def _stage_alpha():
        pltpu.sync_copy(asrc_hbm, shared_asrc)
        pltpu.sync_copy(adst_hbm, shared_adst)

    zero16 = jnp.zeros((16,), jnp.float32)

    @pl.loop(0, 128)
    def _zero_rows(r):
        for m in range(8):
            rows_v[0, r, pl.ds(m * 16, 16)] = zero16
        for k in range(8):
            w_v[0, pl.ds(k * 16, 16)] = zero16

    for t in range(RPT // 128):
        pltpu.sync_copy(rows_v.at[0],
                        shared_out.at[pl.ds(s * RPT + t * 128, 128)])
        pltpu.sync_copy(w_v.at[0],
                        shared_dn.at[pl.ds(s * RPT + t * 128, 128)])
    plsc.subcore_barrier()


    def prep(jn, slot, pb):
        pltpu.sync_copy(edges_hbm.at[pl.ds(2 * (wid * GPT + jn), 2)],
                        e_v.at[pl.ds(2 * slot, 2)])
        pltpu.sync_copy(shared_asrc.at[e_v.at[2 * slot]], av_v)
        pltpu.sync_copy(shared_adst.at[e_v.at[2 * slot + 1]], bv_v)
        for k in range(8):
            e = av_v[pl.ds(k * 16, 16)] + bv_v[pl.ds(k * 16, 16)]
            e = jnp.where(e > 0.0, e, 0.2 * e)
            w_v[pb, pl.ds(k * 16, 16)] = jnp.exp(e)

    def dn_scatter(slot, pb):
        pltpu.sync_copy(w_v.at[pb], shared_dn.at[e_v.at[2 * slot + 1]],
                        add=True)

    def start_gather(slot, rb):
        pltpu.async_copy(h_hbm.at[e_v.at[2 * slot]], rows_v.at[rb],
                         sem_g[rb])

    def wait_gather(slot, rb):
        pltpu.make_async_copy(h_hbm.at[e_v.at[2 * slot]], rows_v.at[rb],
                              sem_g[rb]).wait()

    def start_scatter(slot, rb):
        pltpu.async_copy(rows_v.at[rb], shared_out.at[e_v.at[2 * slot + 1]],
                         sem_s[rb], add=True)

    def wait_scatter(slot, rb):
        pltpu.make_async_copy(rows_v.at[rb],
                              shared_out.at[e_v.at[2 * slot + 1]],
                              sem_s[rb]).wait()

    def scale(rb, pb):
        @pl.loop(0, 8)
        def _scale(k):
            wvec = w_v[pb, pl.ds(k * 16, 16)]
            for l in range(16):
                wb = lax.broadcast(wvec[l], (16,))
                r = k * 16 + l
                for m in range(8):
                    rows_v[rb, r, pl.ds(m * 16, 16)] = (
                        rows_v[rb, r, pl.ds(m * 16, 16)] * wb)

    def body(j, b, sl, sln, slp, first=False):
        wait_gather(sl, b)
        dn_scatter(sl, b)
        prep(j + 1, sln, 1 - b)
        if not first:
            wait_scatter(slp, 1 - b)
        start_gather(sln, 1 - b)
        scale(b, b)
        start_scatter(sl, b)

    prep(0, 0, 0)
    start_gather(0, 0)
    body(0, 0, 0, 1, 3, first=True)
    body(1, 1, 1, 2, 0)
    body(2, 0, 2, 3, 1)
    body(3, 1, 3, 0, 2)

    @pl.loop(0, (GPT - 4) // 4)
    def _edges(q):
        j = 4 + q * 4
        body(j, 0, 0, 1, 3)
        body(j + 1, 1, 1, 2, 0)
        body(j + 2, 0, 2, 3, 1)
        body(j + 3, 1, 3, 0, 2)

    wait_gather(0, 0)
    wait_scatter(3, 1)

    plsc.subcore_barrier()

    for t in range(RPT // 128):
        pltpu.sync_copy(shared_out.at[pl.ds(s * RPT + t * 128, 128)],
                        outp_hbm.at[c, pl.ds(s * RPT + t * 128, 128)])
        pltpu.sync_copy(shared_dn.at[pl.ds(s * RPT + t * 128, 128)],
                        dnp_hbm.at[c, s * (RPT // 128) + t])


def _build_sc_edge(interpret=False):
    mesh = plsc.VectorSubcoreMesh(core_axis_name="c", subcore_axis_name="s",
                                  num_cores=NC, num_subcores=NS)
    return pl.kernel(
        _sc_edge_body,
        out_type=[
            jax.ShapeDtypeStruct((NC, NPAD, D), jnp.float32),
            jax.ShapeDtypeStruct((NC, NPAD // 128, 128), jnp.float32),
        ],
        mesh=mesh,
        scratch_types=[
            pltpu.VMEM((8, 128), jnp.int32),
            pltpu.VMEM((128,), jnp.float32),
            pltpu.VMEM((128,), jnp.float32),
            pltpu.VMEM((2, 128), jnp.float32),
            pltpu.VMEM((2, 128, D), jnp.float32),
            pltpu.VMEM_SHARED((NPAD, D), jnp.float32),
            pltpu.VMEM_SHARED((NPAD,), jnp.float32),
            pltpu.VMEM_SHARED((NPAD,), jnp.float32),
            pltpu.VMEM_SHARED((NPAD,), jnp.float32),
            pltpu.SemaphoreType.DMA,
            pltpu.SemaphoreType.DMA,
            pltpu.SemaphoreType.DMA,
            pltpu.SemaphoreType.DMA,
        ],
        compiler_params=pltpu.CompilerParams(needs_layout_passes=False),
        interpret=interpret,
    )


_BM = 1024
_GRID = NPAD // _BM


def _mm_kernel(x_ref, w_ref, a_ref, h_ref, av_ref):
    xb = x_ref[...]
    h = jnp.dot(xb, w_ref[...], preferred_element_type=jnp.float32)
    h_ref[...] = h
    av_ref[...] = jnp.dot(h, a_ref[...], preferred_element_type=jnp.float32)


def _build_tc_matmul(interpret=False):
    return pl.pallas_call(
        _mm_kernel,
        grid=(_GRID,),
        in_specs=[
            pl.BlockSpec((_BM, D), lambda i: (i, 0)),
            pl.BlockSpec((D, D), lambda i: (0, 0)),
            pl.BlockSpec((D, 128), lambda i: (0, 0)),
        ],
        out_specs=[
            pl.BlockSpec((_BM, D), lambda i: (i, 0)),
            pl.BlockSpec((_BM, 128), lambda i: (i, 0)),
        ],
        out_shape=[
            jax.ShapeDtypeStruct((NPAD, D), jnp.float32),
            jax.ShapeDtypeStruct((NPAD, 128), jnp.float32),
        ],
        interpret=interpret,
    )


def _dn_rows(dn8):
    nq = lax.broadcasted_iota(jnp.int32, (_BM, _BM // 128), 0) // 128
    kq = lax.broadcasted_iota(jnp.int32, (_BM, _BM // 128), 1)
    r = jnp.where(nq == kq, 1.0, 0.0)
    t = jnp.dot(r, dn8, preferred_element_type=jnp.float32)
    nc = lax.broadcasted_iota(jnp.int32, (_BM, 128), 0) % 128
    cc = lax.broadcasted_iota(jnp.int32, (_BM, 128), 1)
    tm = jnp.where(nc == cc, t, 0.0)
    return jnp.dot(tm, jnp.ones((128, 128), jnp.float32),
                   preferred_element_type=jnp.float32)


def _epi_mm_kernel(p_ref, dn_ref, b_ref, w_ref, a_ref, h_ref, av_ref):
    p = p_ref[0] + p_ref[1]
    dn = _dn_rows(dn_ref[0] + dn_ref[1] + 1e-16)
    y = p / dn + b_ref[...]
    y = jnp.maximum(y, 0.0)
    h = jnp.dot(y, w_ref[...], preferred_element_type=jnp.float32)
    h_ref[...] = h
    av_ref[...] = jnp.dot(h, a_ref[...], preferred_element_type=jnp.float32)


def _build_tc_epi_matmul(interpret=False):
    return pl.pallas_call(
        _epi_mm_kernel,
        grid=(_GRID,),
        in_specs=[
            pl.BlockSpec((NC, _BM, D), lambda i: (0, i, 0)),
            pl.BlockSpec((NC, _BM // 128, 128), lambda i: (0, i, 0)),
            pl.BlockSpec((1, D), lambda i: (0, 0)),
            pl.BlockSpec((D, D), lambda i: (0, 0)),
            pl.BlockSpec((D, 128), lambda i: (0, 0)),
        ],
        out_specs=[
            pl.BlockSpec((_BM, D), lambda i: (i, 0)),
            pl.BlockSpec((_BM, 128), lambda i: (i, 0)),
        ],
        out_shape=[
            jax.ShapeDtypeStruct((NPAD, D), jnp.float32),
            jax.ShapeDtypeStruct((NPAD, 128), jnp.float32),
        ],
        interpret=interpret,
    )


def _epi_kernel(p_ref, dn_ref, b_ref, o_ref):
    p = p_ref[0] + p_ref[1]
    dn = _dn_rows(dn_ref[0] + dn_ref[1] + 1e-16)
    o_ref[...] = p / dn + b_ref[...]


def _build_tc_epilogue(interpret=False):
    return pl.pallas_call(
        _epi_kernel,
        grid=(_GRID,),
        in_specs=[
            pl.BlockSpec((NC, _BM, D), lambda i: (0, i, 0)),
            pl.BlockSpec((NC, _BM // 128, 128), lambda i: (0, i, 0)),
            pl.BlockSpec((1, D), lambda i: (0, 0)),
        ],
        out_specs=pl.BlockSpec((_BM, D), lambda i: (i, 0)),
        out_shape=jax.ShapeDtypeStruct((NPAD, D), jnp.float32),
        interpret=interpret,
    )


_build_sc_edge = functools.lru_cache(maxsize=None)(_build_sc_edge)
_build_tc_matmul = functools.lru_cache(maxsize=None)(_build_tc_matmul)
_build_tc_epi_matmul = functools.lru_cache(maxsize=None)(_build_tc_epi_matmul)
_build_tc_epilogue = functools.lru_cache(maxsize=None)(_build_tc_epilogue)


@jax.jit
def kernel(x, edge_index, W1, a1_src, a1_dst, b1, W2, a2_src, a2_dst, b2):
    src = edge_index[0].astype(jnp.int32)
    dst = edge_index[1].astype(jnp.int32)
    pad_idx = N + (jnp.arange(EPAD - E, dtype=jnp.int32) % (NPAD - N))
    src2d = jnp.concatenate([src, pad_idx]).reshape(EDGE_ROWS, 128)
    dst2d = jnp.concatenate([dst, pad_idx]).reshape(EDGE_ROWS, 128)
    e3 = jnp.stack([src2d, dst2d], axis=1)
    extra = jnp.broadcast_to(N + (jnp.arange(128, dtype=jnp.int32)
                                  % (NPAD - N)), (8, 2, 128))
    e3 = jnp.concatenate([e3, extra], axis=0)
    e3 = e3.reshape(2 * (EDGE_ROWS + 8), 128)

    xp = jnp.zeros((NPAD, D), jnp.float32).at[:N].set(x)
    A1 = (jnp.zeros((D, 128), jnp.float32)
          .at[:, 0].set(a1_src).at[:, 1].set(a1_dst))
    A2 = (jnp.zeros((D, 128), jnp.float32)
          .at[:, 0].set(a2_src).at[:, 1].set(a2_dst))

    sc_edge = _build_sc_edge()
    h1, av1 = _build_tc_matmul()(xp, W1, A1)
    outp1, dnp1 = sc_edge(h1, av1[:, 0], av1[:, 1], e3)
    h2, av2 = _build_tc_epi_matmul()(outp1, dnp1, b1.reshape(1, D), W2, A2)
    outp2, dnp2 = sc_edge(h2, av2[:, 0], av2[:, 1], e3)
    out = _build_tc_epilogue()(outp2, dnp2, b2.reshape(1, D))
    return out[:N]


# speedup vs baseline: 44.3172x; 1.0724x over previous
import functools

import jax
import jax.numpy as jnp
from jax import lax
from jax.experimental import pallas as pl
from jax.experimental.pallas import tpu as pltpu
from jax.experimental.pallas import tpu_sc as plsc

N = 10000
E = 320000
D = 128

NPAD = 10240
EPAD = 327680
NC = 2
NS = 16
GPT = EPAD // (NC * NS * 128)
EDGE_ROWS = EPAD // 128
RPT = NPAD // NS


NSB = GPT // 4


def _sc_edge_body(h_hbm, asrc_hbm, adst_hbm, src_hbm, dst_hbm,
                  outp_hbm, dnp_hbm,
                  src_sb, dst_sb, av_sb, bv_sb, w_sb, rows_v,
                  shared_out, shared_dn, shared_asrc, shared_adst,
                  sem_g0, sem_g1, sem_s0, sem_s1):
    c = lax.axis_index("c")
    s = lax.axis_index("s")
    wid = c * NS + s
    sem_g = (sem_g0, sem_g1)
    sem_s = (sem_s0, sem_s1)

    @pl.when(s == 0)
    def _stage_alpha():
        pltpu.sync_copy(asrc_hbm, shared_asrc)
        pltpu.sync_copy(adst_hbm, shared_adst)

    zero16 = jnp.zeros((16,), jnp.float32)

    @pl.loop(0, 128)
    def _zero_rows(r):
        for m in range(8):
            rows_v[0, r, pl.ds(m * 16, 16)] = zero16

    for t in range(RPT // 128):
        pltpu.sync_copy(rows_v.at[0],
                        shared_out.at[pl.ds(s * RPT + t * 128, 128)])
        pltpu.sync_copy(rows_v.at[0, 0],
                        shared_dn.at[pl.ds(s * RPT + t * 128, 128)])
    plsc.subcore_barrier()


    def prep_sb(t1, pbn):
        g0 = wid * GPT + 4 * t1
        pltpu.sync_copy(src_hbm.at[pl.ds(g0, 4)], src_sb.at[pbn])
        pltpu.sync_copy(dst_hbm.at[pl.ds(g0, 4)], dst_sb.at[pbn])
        for o in range(4):
            pltpu.sync_copy(shared_asrc.at[src_sb.at[pbn, o]], av_sb.at[o])
            pltpu.sync_copy(shared_adst.at[dst_sb.at[pbn, o]], bv_sb.at[o])
        for o in range(4):
            for k in range(8):
                e = (av_sb[o, pl.ds(k * 16, 16)]
                     + bv_sb[o, pl.ds(k * 16, 16)])
                e = jnp.where(e > 0.0, e, 0.2 * e)
                w_sb[pbn, o, pl.ds(k * 16, 16)] = jnp.exp(e)

        @pl.when(t1 < NSB)
        def _dn():
            for o in range(4):
                pltpu.sync_copy(w_sb.at[pbn, o],
                                shared_dn.at[dst_sb.at[pbn, o]], add=True)

    def start_gather(pb, o, rb):
        pltpu.async_copy(h_hbm.at[src_sb.at[pb, o]], rows_v.at[rb],
                         sem_g[rb])

    def wait_gather(pb, o, rb):
        pltpu.make_async_copy(h_hbm.at[src_sb.at[pb, o]], rows_v.at[rb],
                              sem_g[rb]).wait()

    def start_scatter(pb, o, rb):
        pltpu.async_copy(rows_v.at[rb], shared_out.at[dst_sb.at[pb, o]],
                         sem_s[rb], add=True)

    def wait_scatter(pb, o, rb):
        pltpu.make_async_copy(rows_v.at[rb],
                              shared_out.at[dst_sb.at[pb, o]],
                              sem_s[rb]).wait()

    def scale(rb, pb, o):
        @pl.loop(0, 8)
        def _scale(k):
            wvec = w_sb[pb, o, pl.ds(k * 16, 16)]
            for l in range(16):
                wb = lax.broadcast(wvec[l], (16,))
                r = k * 16 + l
                for m in range(8):
                    rows_v[rb, r, pl.ds(m * 16, 16)] = (
                        rows_v[rb, r, pl.ds(m * 16, 16)] * wb)

    def gbody(q, u):
        pb, o, rb = u // 4, u % 4, u % 2
        wait_gather(pb, o, rb)
        if o == 2:
            prep_sb(2 * q + u // 4 + 1, (u // 4 + 1) % 2)
        pu = (u - 1) % 8
        if u == 0:
            @pl.when(q > 0)
            def _ws():
                wait_scatter(pu // 4, pu % 4, pu % 2)
        else:
            wait_scatter(pu // 4, pu % 4, pu % 2)
        nu = (u + 1) % 8
        start_gather(nu // 4, nu % 4, nu % 2)
        scale(rb, pb, o)
        start_scatter(pb, o, rb)

    prep_sb(0, 0)
    start_gather(0, 0, 0)

    @pl.loop(0, GPT // 8)
    def _edges(q):
        for u in range(8):
            gbody(q, u)

    wait_gather(0, 0, 0)
    wait_scatter(1, 3, 1)

    plsc.subcore_barrier()

    for t in range(RPT // 128):
        pltpu.sync_copy(shared_out.at[pl.ds(s * RPT + t * 128, 128)],
                        outp_hbm.at[c, pl.ds(s * RPT + t * 128, 128)])
        pltpu.sync_copy(shared_dn.at[pl.ds(s * RPT + t * 128, 128)],
                        dnp_hbm.at[c, s * (RPT // 128) + t])


def _build_sc_edge(interpret=False):
    mesh = plsc.VectorSubcoreMesh(core_axis_name="c", subcore_axis_name="s",
                                  num_cores=NC, num_subcores=NS)
    return pl.kernel(
        _sc_edge_body,
        out_type=[
            jax.ShapeDtypeStruct((NC, NPAD, D), jnp.float32),
            jax.ShapeDtypeStruct((NC, NPAD // 128, 128), jnp.float32),
        ],
        mesh=mesh,
        scratch_types=[
            pltpu.VMEM((2, 4, 128), jnp.int32),
            pltpu.VMEM((2, 4, 128), jnp.int32),
            pltpu.VMEM((4, 128), jnp.float32),
            pltpu.VMEM((4, 128), jnp.float32),
            pltpu.VMEM((2, 4, 128), jnp.float32),
            pltpu.VMEM((2, 128, D), jnp.float32),
            pltpu.VMEM_SHARED((NPAD, D), jnp.float32),
            pltpu.VMEM_SHARED((NPAD,), jnp.float32),
            pltpu.VMEM_SHARED((NPAD,), jnp.float32),
            pltpu.VMEM_SHARED((NPAD,), jnp.float32),
            pltpu.SemaphoreType.DMA,
            pltpu.SemaphoreType.DMA,
            pltpu.SemaphoreType.DMA,
            pltpu.SemaphoreType.DMA,
        ],
        compiler_params=pltpu.CompilerParams(needs_layout_passes=False),
        interpret=interpret,
    )


_BM = 1024
_GRID = NPAD // _BM


def _mm_kernel(x_ref, w_ref, a_ref, h_ref, av_ref):
    xb = x_ref[...]
    h = jnp.dot(xb, w_ref[...], preferred_element_type=jnp.float32)
    h_ref[...] = h
    av_ref[...] = jnp.dot(h, a_ref[...], preferred_element_type=jnp.float32)


def _build_tc_matmul(interpret=False):
    return pl.pallas_call(
        _mm_kernel,
        grid=(_GRID,),
        in_specs=[
            pl.BlockSpec((_BM, D), lambda i: (i, 0)),
            pl.BlockSpec((D, D), lambda i: (0, 0)),
            pl.BlockSpec((D, 128), lambda i: (0, 0)),
        ],
        out_specs=[
            pl.BlockSpec((_BM, D), lambda i: (i, 0)),
            pl.BlockSpec((_BM, 128), lambda i: (i, 0)),
        ],
        out_shape=[
            jax.ShapeDtypeStruct((NPAD, D), jnp.float32),
            jax.ShapeDtypeStruct((NPAD, 128), jnp.float32),
        ],
        interpret=interpret,
    )


def _dn_rows(dn8):
    nq = lax.broadcasted_iota(jnp.int32, (_BM, _BM // 128), 0) // 128
    kq = lax.broadcasted_iota(jnp.int32, (_BM, _BM // 128), 1)
    r = jnp.where(nq == kq, 1.0, 0.0)
    t = jnp.dot(r, dn8, preferred_element_type=jnp.float32)
    nc = lax.broadcasted_iota(jnp.int32, (_BM, 128), 0) % 128
    cc = lax.broadcasted_iota(jnp.int32, (_BM, 128), 1)
    tm = jnp.where(nc == cc, t, 0.0)
    return jnp.dot(tm, jnp.ones((128, 128), jnp.float32),
                   preferred_element_type=jnp.float32)


def _epi_mm_kernel(p_ref, dn_ref, b_ref, w_ref, a_ref, h_ref, av_ref):
    p = p_ref[0] + p_ref[1]
    dn = _dn_rows(dn_ref[0] + dn_ref[1] + 1e-16)
    y = p / dn + b_ref[...]
    y = jnp.maximum(y, 0.0)
    h = jnp.dot(y, w_ref[...], preferred_element_type=jnp.float32)
    h_ref[...] = h
    av_ref[...] = jnp.dot(h, a_ref[...], preferred_element_type=jnp.float32)


def _build_tc_epi_matmul(interpret=False):
    return pl.pallas_call(
        _epi_mm_kernel,
        grid=(_GRID,),
        in_specs=[
            pl.BlockSpec((NC, _BM, D), lambda i: (0, i, 0)),
            pl.BlockSpec((NC, _BM // 128, 128), lambda i: (0, i, 0)),
            pl.BlockSpec((1, D), lambda i: (0, 0)),
            pl.BlockSpec((D, D), lambda i: (0, 0)),
            pl.BlockSpec((D, 128), lambda i: (0, 0)),
        ],
        out_specs=[
            pl.BlockSpec((_BM, D), lambda i: (i, 0)),
            pl.BlockSpec((_BM, 128), lambda i: (i, 0)),
        ],
        out_shape=[
            jax.ShapeDtypeStruct((NPAD, D), jnp.float32),
            jax.ShapeDtypeStruct((NPAD, 128), jnp.float32),
        ],
        interpret=interpret,
    )


def _epi_kernel(p_ref, dn_ref, b_ref, o_ref):
    p = p_ref[0] + p_ref[1]
    dn = _dn_rows(dn_ref[0] + dn_ref[1] + 1e-16)
    o_ref[...] = p / dn + b_ref[...]


def _build_tc_epilogue(interpret=False):
    return pl.pallas_call(
        _epi_kernel,
        grid=(_GRID,),
        in_specs=[
            pl.BlockSpec((NC, _BM, D), lambda i: (0, i, 0)),
            pl.BlockSpec((NC, _BM // 128, 128), lambda i: (0, i, 0)),
            pl.BlockSpec((1, D), lambda i: (0, 0)),
        ],
        out_specs=pl.BlockSpec((_BM, D), lambda i: (i, 0)),
        out_shape=jax.ShapeDtypeStruct((NPAD, D), jnp.float32),
        interpret=interpret,
    )


_build_sc_edge = functools.lru_cache(maxsize=None)(_build_sc_edge)
_build_tc_matmul = functools.lru_cache(maxsize=None)(_build_tc_matmul)
_build_tc_epi_matmul = functools.lru_cache(maxsize=None)(_build_tc_epi_matmul)
_build_tc_epilogue = functools.lru_cache(maxsize=None)(_build_tc_epilogue)


@jax.jit
def kernel(x, edge_index, W1, a1_src, a1_dst, b1, W2, a2_src, a2_dst, b2):
    src = edge_index[0].astype(jnp.int32)
    dst = edge_index[1].astype(jnp.int32)
    pad_idx = N + (jnp.arange(EPAD - E, dtype=jnp.int32) % (NPAD - N))
    src2d = jnp.concatenate([src, pad_idx]).reshape(EDGE_ROWS, 128)
    dst2d = jnp.concatenate([dst, pad_idx]).reshape(EDGE_ROWS, 128)
    extra = jnp.broadcast_to(N + (jnp.arange(128, dtype=jnp.int32)
                                  % (NPAD - N)), (8, 128))
    src2d = jnp.concatenate([src2d, extra])
    dst2d = jnp.concatenate([dst2d, extra])

    xp = jnp.zeros((NPAD, D), jnp.float32).at[:N].set(x)
    A1 = (jnp.zeros((D, 128), jnp.float32)
          .at[:, 0].set(a1_src).at[:, 1].set(a1_dst))
    A2 = (jnp.zeros((D, 128), jnp.float32)
          .at[:, 0].set(a2_src).at[:, 1].set(a2_dst))

    sc_edge = _build_sc_edge()
    h1, av1 = _build_tc_matmul()(xp, W1, A1)
    outp1, dnp1 = sc_edge(h1, av1[:, 0], av1[:, 1], src2d, dst2d)
    h2, av2 = _build_tc_epi_matmul()(outp1, dnp1, b1.reshape(1, D), W2, A2)
    outp2, dnp2 = sc_edge(h2, av2[:, 0], av2[:, 1], src2d, dst2d)
    out = _build_tc_epilogue()(outp2, dnp2, b2.reshape(1, D))
    return out[:N]


# speedup vs baseline: 50.7532x; 1.1452x over previous
import functools

import jax
import jax.numpy as jnp
from jax import lax
from jax.experimental import pallas as pl
from jax.experimental.pallas import tpu as pltpu
from jax.experimental.pallas import tpu_sc as plsc

N = 10000
E = 320000
D = 128

NPAD = 10240
EPAD = 327680
NC = 2
NS = 16
GPT = EPAD // (NC * NS * 128)
EDGE_ROWS = EPAD // 128
RPT = NPAD // NS


NSB = GPT // 4


def _sc_edge_body(h_hbm, asrc_hbm, adst_hbm, src_hbm, dst_hbm,
                  outp_hbm, dnp_hbm,
                  src_sb, dst_sb, av_sb, bv_sb, w_sb, rows_v,
                  shared_out, shared_dn, shared_asrc, shared_adst,
                  sem_g0, sem_g1, sem_s0, sem_s1, sem_p, sem_d0, sem_d1):
    c = lax.axis_index("c")
    s = lax.axis_index("s")
    wid = c * NS + s
    sem_g = (sem_g0, sem_g1)
    sem_s = (sem_s0, sem_s1)
    sem_d = (sem_d0, sem_d1)

    @pl.when(s == 0)
    def _stage_alpha():
        pltpu.sync_copy(asrc_hbm, shared_asrc)
        pltpu.sync_copy(adst_hbm, shared_adst)

    zero16 = jnp.zeros((16,), jnp.float32)

    @pl.loop(0, 128)
    def _zero_rows(r):
        for m in range(8):
            rows_v[0, r, pl.ds(m * 16, 16)] = zero16

    for t in range(RPT // 128):
        pltpu.sync_copy(rows_v.at[0],
                        shared_out.at[pl.ds(s * RPT + t * 128, 128)])
        pltpu.sync_copy(rows_v.at[0, 0],
                        shared_dn.at[pl.ds(s * RPT + t * 128, 128)])
    plsc.subcore_barrier()


    def prep_sb(t1, pbn, drain=True):
        if drain:
            @pl.when(t1 >= 2)
            def _drain():
                for o in range(4):
                    pltpu.make_async_copy(w_sb.at[pbn, o],
                                          shared_dn.at[dst_sb.at[pbn, o]],
                                          sem_d[pbn]).wait()
        g0 = wid * GPT + 4 * t1
        d1 = pltpu.async_copy(src_hbm.at[pl.ds(g0, 4)], src_sb.at[pbn],
                              sem_p)
        d2 = pltpu.async_copy(dst_hbm.at[pl.ds(g0, 4)], dst_sb.at[pbn],
                              sem_p)
        d1.wait()
        d2.wait()
        ds = []
        for o in range(4):
            ds.append(pltpu.async_copy(shared_asrc.at[src_sb.at[pbn, o]],
                                       av_sb.at[o], sem_p))
            ds.append(pltpu.async_copy(shared_adst.at[dst_sb.at[pbn, o]],
                                       bv_sb.at[o], sem_p))
        for d in ds:
            d.wait()
        for o in range(4):
            for k in range(8):
                e = (av_sb[o, pl.ds(k * 16, 16)]
                     + bv_sb[o, pl.ds(k * 16, 16)])
                e = jnp.where(e > 0.0, e, 0.2 * e)
                w_sb[pbn, o, pl.ds(k * 16, 16)] = jnp.exp(e)

        @pl.when(t1 < NSB)
        def _dn():
            for o in range(4):
                pltpu.async_copy(w_sb.at[pbn, o],
                                 shared_dn.at[dst_sb.at[pbn, o]],
                                 sem_d[pbn], add=True)

    def start_gather(pb, o, rb):
        pltpu.async_copy(h_hbm.at[src_sb.at[pb, o]], rows_v.at[rb],
                         sem_g[rb])

    def wait_gather(pb, o, rb):
        pltpu.make_async_copy(h_hbm.at[src_sb.at[pb, o]], rows_v.at[rb],
                              sem_g[rb]).wait()

    def start_scatter(pb, o, rb):
        pltpu.async_copy(rows_v.at[rb], shared_out.at[dst_sb.at[pb, o]],
                         sem_s[rb], add=True)

    def wait_scatter(pb, o, rb):
        pltpu.make_async_copy(rows_v.at[rb],
                              shared_out.at[dst_sb.at[pb, o]],
                              sem_s[rb]).wait()

    def scale(rb, pb, o):
        @pl.loop(0, 8)
        def _scale(k):
            wvec = w_sb[pb, o, pl.ds(k * 16, 16)]
            for l in range(16):
                wb = lax.broadcast(wvec[l], (16,))
                r = k * 16 + l
                for m in range(8):
                    rows_v[rb, r, pl.ds(m * 16, 16)] = (
                        rows_v[rb, r, pl.ds(m * 16, 16)] * wb)

    def gbody(q, u):
        pb, o, rb = u // 4, u % 4, u % 2
        wait_gather(pb, o, rb)
        if o == 2:
            prep_sb(2 * q + u // 4 + 1, (u // 4 + 1) % 2)
        pu = (u - 1) % 8
        if u == 0:
            @pl.when(q > 0)
            def _ws():
                wait_scatter(pu // 4, pu % 4, pu % 2)
        else:
            wait_scatter(pu // 4, pu % 4, pu % 2)
        nu = (u + 1) % 8
        start_gather(nu // 4, nu % 4, nu % 2)
        scale(rb, pb, o)
        start_scatter(pb, o, rb)

    prep_sb(0, 0, drain=False)
    start_gather(0, 0, 0)

    @pl.loop(0, GPT // 8)
    def _edges(q):
        for u in range(8):
            gbody(q, u)

    wait_gather(0, 0, 0)
    wait_scatter(1, 3, 1)
    for o in range(4):
        pltpu.make_async_copy(w_sb.at[1, o],
                              shared_dn.at[dst_sb.at[1, o]],
                              sem_d[1]).wait()

    plsc.subcore_barrier()

    for t in range(RPT // 128):
        pltpu.sync_copy(shared_out.at[pl.ds(s * RPT + t * 128, 128)],
                        outp_hbm.at[c, pl.ds(s * RPT + t * 128, 128)])
        pltpu.sync_copy(shared_dn.at[pl.ds(s * RPT + t * 128, 128)],
                        dnp_hbm.at[c, s * (RPT // 128) + t])


def _build_sc_edge(interpret=False):
    mesh = plsc.VectorSubcoreMesh(core_axis_name="c", subcore_axis_name="s",
                                  num_cores=NC, num_subcores=NS)
    return pl.kernel(
        _sc_edge_body,
        out_type=[
            jax.ShapeDtypeStruct((NC, NPAD, D), jnp.float32),
            jax.ShapeDtypeStruct((NC, NPAD // 128, 128), jnp.float32),
        ],
        mesh=mesh,
        scratch_types=[
            pltpu.VMEM((2, 4, 128), jnp.int32),
            pltpu.VMEM((2, 4, 128), jnp.int32),
            pltpu.VMEM((4, 128), jnp.float32),
            pltpu.VMEM((4, 128), jnp.float32),
            pltpu.VMEM((2, 4, 128), jnp.float32),
            pltpu.VMEM((2, 128, D), jnp.float32),
            pltpu.VMEM_SHARED((NPAD, D), jnp.float32),
            pltpu.VMEM_SHARED((NPAD,), jnp.float32),
            pltpu.VMEM_SHARED((NPAD,), jnp.float32),
            pltpu.VMEM_SHARED((NPAD,), jnp.float32),
            pltpu.SemaphoreType.DMA,
            pltpu.SemaphoreType.DMA,
            pltpu.SemaphoreType.DMA,
            pltpu.SemaphoreType.DMA,
            pltpu.SemaphoreType.DMA,
            pltpu.SemaphoreType.DMA,
            pltpu.SemaphoreType.DMA,
        ],
        compiler_params=pltpu.CompilerParams(needs_layout_passes=False),
        interpret=interpret,
    )


_BM = 1024
_GRID = NPAD // _BM


def _mm_kernel(x_ref, w_ref, a_ref, h_ref, av_ref):
    xb = x_ref[...]
    h = jnp.dot(xb, w_ref[...], preferred_element_type=jnp.float32)
    h_ref[...] = h
    av_ref[...] = jnp.dot(h, a_ref[...], preferred_element_type=jnp.float32)


def _build_tc_matmul(interpret=False):
    return pl.pallas_call(
        _mm_kernel,
        grid=(_GRID,),
        in_specs=[
            pl.BlockSpec((_BM, D), lambda i: (i, 0)),
            pl.BlockSpec((D, D), lambda i: (0, 0)),
            pl.BlockSpec((D, 128), lambda i: (0, 0)),
        ],
        out_specs=[
            pl.BlockSpec((_BM, D), lambda i: (i, 0)),
            pl.BlockSpec((_BM, 128), lambda i: (i, 0)),
        ],
        out_shape=[
            jax.ShapeDtypeStruct((NPAD, D), jnp.float32),
            jax.ShapeDtypeStruct((NPAD, 128), jnp.float32),
        ],
        interpret=interpret,
    )


def _dn_rows(dn8):
    nq = lax.broadcasted_iota(jnp.int32, (_BM, _BM // 128), 0) // 128
    kq = lax.broadcasted_iota(jnp.int32, (_BM, _BM // 128), 1)
    r = jnp.where(nq == kq, 1.0, 0.0)
    t = jnp.dot(r, dn8, preferred_element_type=jnp.float32)
    nc = lax.broadcasted_iota(jnp.int32, (_BM, 128), 0) % 128
    cc = lax.broadcasted_iota(jnp.int32, (_BM, 128), 1)
    tm = jnp.where(nc == cc, t, 0.0)
    return jnp.dot(tm, jnp.ones((128, 128), jnp.float32),
                   preferred_element_type=jnp.float32)


def _epi_mm_kernel(p_ref, dn_ref, b_ref, w_ref, a_ref, h_ref, av_ref):
    p = p_ref[0] + p_ref[1]
    dn = _dn_rows(dn_ref[0] + dn_ref[1] + 1e-16)
    y = p / dn + b_ref[...]
    y = jnp.maximum(y, 0.0)
    h = jnp.dot(y, w_ref[...], preferred_element_type=jnp.float32)
    h_ref[...] = h
    av_ref[...] = jnp.dot(h, a_ref[...], preferred_element_type=jnp.float32)


def _build_tc_epi_matmul(interpret=False):
    return pl.pallas_call(
        _epi_mm_kernel,
        grid=(_GRID,),
        in_specs=[
            pl.BlockSpec((NC, _BM, D), lambda i: (0, i, 0)),
            pl.BlockSpec((NC, _BM // 128, 128), lambda i: (0, i, 0)),
            pl.BlockSpec((1, D), lambda i: (0, 0)),
            pl.BlockSpec((D, D), lambda i: (0, 0)),
            pl.BlockSpec((D, 128), lambda i: (0, 0)),
        ],
        out_specs=[
            pl.BlockSpec((_BM, D), lambda i: (i, 0)),
            pl.BlockSpec((_BM, 128), lambda i: (i, 0)),
        ],
        out_shape=[
            jax.ShapeDtypeStruct((NPAD, D), jnp.float32),
            jax.ShapeDtypeStruct((NPAD, 128), jnp.float32),
        ],
        interpret=interpret,
    )


def _epi_kernel(p_ref, dn_ref, b_ref, o_ref):
    p = p_ref[0] + p_ref[1]
    dn = _dn_rows(dn_ref[0] + dn_ref[1] + 1e-16)
    o_ref[...] = p / dn + b_ref[...]


def _build_tc_epilogue(interpret=False):
    return pl.pallas_call(
        _epi_kernel,
        grid=(_GRID,),
        in_specs=[
            pl.BlockSpec((NC, _BM, D), lambda i: (0, i, 0)),
            pl.BlockSpec((NC, _BM // 128, 128), lambda i: (0, i, 0)),
            pl.BlockSpec((1, D), lambda i: (0, 0)),
        ],
        out_specs=pl.BlockSpec((_BM, D), lambda i: (i, 0)),
        out_shape=jax.ShapeDtypeStruct((NPAD, D), jnp.float32),
        interpret=interpret,
    )


_build_sc_edge = functools.lru_cache(maxsize=None)(_build_sc_edge)
_build_tc_matmul = functools.lru_cache(maxsize=None)(_build_tc_matmul)
_build_tc_epi_matmul = functools.lru_cache(maxsize=None)(_build_tc_epi_matmul)
_build_tc_epilogue = functools.lru_cache(maxsize=None)(_build_tc_epilogue)


@jax.jit
def kernel(x, edge_index, W1, a1_src, a1_dst, b1, W2, a2_src, a2_dst, b2):
    src = edge_index[0].astype(jnp.int32)
    dst = edge_index[1].astype(jnp.int32)
    pad_idx = N + (jnp.arange(EPAD - E, dtype=jnp.int32) % (NPAD - N))
    src2d = jnp.concatenate([src, pad_idx]).reshape(EDGE_ROWS, 128)
    dst2d = jnp.concatenate([dst, pad_idx]).reshape(EDGE_ROWS, 128)
    extra = jnp.broadcast_to(N + (jnp.arange(128, dtype=jnp.int32)
                                  % (NPAD - N)), (8, 128))
    src2d = jnp.concatenate([src2d, extra])
    dst2d = jnp.concatenate([dst2d, extra])

    xp = jnp.zeros((NPAD, D), jnp.float32).at[:N].set(x)
    A1 = (jnp.zeros((D, 128), jnp.float32)
          .at[:, 0].set(a1_src).at[:, 1].set(a1_dst))
    A2 = (jnp.zeros((D, 128), jnp.float32)
          .at[:, 0].set(a2_src).at[:, 1].set(a2_dst))

    sc_edge = _build_sc_edge()
    h1, av1 = _build_tc_matmul()(xp, W1, A1)
    outp1, dnp1 = sc_edge(h1, av1[:, 0], av1[:, 1], src2d, dst2d)
    h2, av2 = _build_tc_epi_matmul()(outp1, dnp1, b1.reshape(1, D), W2, A2)
    outp2, dnp2 = sc_edge(h2, av2[:, 0], av2[:, 1], src2d, dst2d)
    out = _build_tc_epilogue()(outp2, dnp2, b2.reshape(1, D))
    return out[:N]


# speedup vs baseline: 52.2506x; 1.0295x over previous
import functools

import jax
import jax.numpy as jnp
from jax import lax
from jax.experimental import pallas as pl
from jax.experimental.pallas import tpu as pltpu
from jax.experimental.pallas import tpu_sc as plsc

N = 10000
E = 320000
D = 128

NPAD = 10240
EPAD = 327680
NC = 2
NS = 16
GPT = EPAD // (NC * NS * 128)
EDGE_ROWS = EPAD // 128
RPT = NPAD // NS


NSB = GPT // 4


def _sc_edge_body(h_hbm, asrc_hbm, adst_hbm, src_hbm, dst_hbm,
                  outp_hbm, dnp_hbm,
                  src_sb, dst_sb, av_sb, bv_sb, w_sb, rows_v,
                  shared_out, shared_dn, shared_asrc, shared_adst,
                  sem_g0, sem_g1, sem_s0, sem_s1, sem_p, sem_d0, sem_d1):
    c = lax.axis_index("c")
    s = lax.axis_index("s")
    wid = c * NS + s
    sem_g = (sem_g0, sem_g1)
    sem_s = (sem_s0, sem_s1)
    sem_d = (sem_d0, sem_d1)

    @pl.when(s == 0)
    def _stage_alpha():
        pltpu.sync_copy(asrc_hbm, shared_asrc)
        pltpu.sync_copy(adst_hbm, shared_adst)

    zero16 = jnp.zeros((16,), jnp.float32)

    @pl.loop(0, 128)
    def _zero_rows(r):
        for m in range(8):
            rows_v[0, r, pl.ds(m * 16, 16)] = zero16

    for t in range(RPT // 128):
        pltpu.sync_copy(rows_v.at[0],
                        shared_out.at[pl.ds(s * RPT + t * 128, 128)])
        pltpu.sync_copy(rows_v.at[0, 0],
                        shared_dn.at[pl.ds(s * RPT + t * 128, 128)])
    plsc.subcore_barrier()


    def prep_sb(t1, pbn, drain=True):
        if drain:
            @pl.when(t1 >= 2)
            def _drain():
                for o in range(4):
                    pltpu.make_async_copy(w_sb.at[pbn, o],
                                          shared_dn.at[dst_sb.at[pbn, o]],
                                          sem_d[pbn]).wait()
        g0 = wid * GPT + 4 * t1
        d1 = pltpu.async_copy(src_hbm.at[pl.ds(g0, 4)], src_sb.at[pbn],
                              sem_p)
        d2 = pltpu.async_copy(dst_hbm.at[pl.ds(g0, 4)], dst_sb.at[pbn],
                              sem_p)
        d1.wait()
        d2.wait()
        ds = []
        for o in range(4):
            ds.append(pltpu.async_copy(shared_asrc.at[src_sb.at[pbn, o]],
                                       av_sb.at[o], sem_p))
            ds.append(pltpu.async_copy(shared_adst.at[dst_sb.at[pbn, o]],
                                       bv_sb.at[o], sem_p))
        for d in ds:
            d.wait()
        for o in range(4):
            for k in range(8):
                e = (av_sb[o, pl.ds(k * 16, 16)]
                     + bv_sb[o, pl.ds(k * 16, 16)])
                e = jnp.where(e > 0.0, e, 0.2 * e)
                w_sb[pbn, o, pl.ds(k * 16, 16)] = jnp.exp(e)

        @pl.when(t1 < NSB)
        def _dn():
            for o in range(4):
                pltpu.async_copy(w_sb.at[pbn, o],
                                 shared_dn.at[dst_sb.at[pbn, o]],
                                 sem_d[pbn], add=True)

    def start_gather(pb, o, rb):
        pltpu.async_copy(h_hbm.at[src_sb.at[pb, o]], rows_v.at[rb],
                         sem_g[rb])

    def wait_gather(pb, o, rb):
        pltpu.make_async_copy(h_hbm.at[src_sb.at[pb, o]], rows_v.at[rb],
                              sem_g[rb]).wait()

    def start_scatter(pb, o, rb):
        pltpu.async_copy(rows_v.at[rb], shared_out.at[dst_sb.at[pb, o]],
                         sem_s[rb], add=True)

    def wait_scatter(pb, o, rb):
        pltpu.make_async_copy(rows_v.at[rb],
                              shared_out.at[dst_sb.at[pb, o]],
                              sem_s[rb]).wait()

    def scale(rb, pb, o):
        @pl.loop(0, 8)
        def _scale(k):
            wvec = w_sb[pb, o, pl.ds(k * 16, 16)]
            for l in range(16):
                wb = lax.broadcast(wvec[l], (16,))
                r = k * 16 + l
                for m in range(8):
                    rows_v[rb, r, pl.ds(m * 16, 16)] = (
                        rows_v[rb, r, pl.ds(m * 16, 16)] * wb)

    def gbody(q, u):
        pb, o, rb = u // 4, u % 4, u % 2
        wait_gather(pb, o, rb)
        if o == 2:
            prep_sb(2 * q + u // 4 + 1, (u // 4 + 1) % 2)
        pu = (u - 1) % 8
        if u == 0:
            @pl.when(q > 0)
            def _ws():
                wait_scatter(pu // 4, pu % 4, pu % 2)
        else:
            wait_scatter(pu // 4, pu % 4, pu % 2)
        nu = (u + 1) % 8
        start_gather(nu // 4, nu % 4, nu % 2)
        scale(rb, pb, o)
        start_scatter(pb, o, rb)

    prep_sb(0, 0, drain=False)
    start_gather(0, 0, 0)

    @pl.loop(0, GPT // 8)
    def _edges(q):
        for u in range(8):
            gbody(q, u)

    wait_gather(0, 0, 0)
    wait_scatter(1, 3, 1)
    for o in range(4):
        pltpu.make_async_copy(w_sb.at[1, o],
                              shared_dn.at[dst_sb.at[1, o]],
                              sem_d[1]).wait()

    plsc.subcore_barrier()

    for t in range(RPT // 128):
        pltpu.sync_copy(shared_out.at[pl.ds(s * RPT + t * 128, 128)],
                        outp_hbm.at[c, pl.ds(s * RPT + t * 128, 128)])
        pltpu.sync_copy(shared_dn.at[pl.ds(s * RPT + t * 128, 128)],
                        dnp_hbm.at[c, s * (RPT // 128) + t])


def _build_sc_edge(interpret=False):
    mesh = plsc.VectorSubcoreMesh(core_axis_name="c", subcore_axis_name="s",
                                  num_cores=NC, num_subcores=NS)
    return pl.kernel(
        _sc_edge_body,
        out_type=[
            jax.ShapeDtypeStruct((NC, NPAD, D), jnp.float32),
            jax.ShapeDtypeStruct((NC, NPAD // 128, 128), jnp.float32),
        ],
        mesh=mesh,
        scratch_types=[
            pltpu.VMEM((2, 4, 128), jnp.int32),
            pltpu.VMEM((2, 4, 128), jnp.int32),
            pltpu.VMEM((4, 128), jnp.float32),
            pltpu.VMEM((4, 128), jnp.float32),
            pltpu.VMEM((2, 4, 128), jnp.float32),
            pltpu.VMEM((2, 128, D), jnp.float32),
            pltpu.VMEM_SHARED((NPAD, D), jnp.float32),
            pltpu.VMEM_SHARED((NPAD,), jnp.float32),
            pltpu.VMEM_SHARED((NPAD,), jnp.float32),
            pltpu.VMEM_SHARED((NPAD,), jnp.float32),
            pltpu.SemaphoreType.DMA,
            pltpu.SemaphoreType.DMA,
            pltpu.SemaphoreType.DMA,
            pltpu.SemaphoreType.DMA,
            pltpu.SemaphoreType.DMA,
            pltpu.SemaphoreType.DMA,
            pltpu.SemaphoreType.DMA,
        ],
        compiler_params=pltpu.CompilerParams(needs_layout_passes=False),
        interpret=interpret,
    )


_BM = 1024
_GRID = NPAD // _BM


def _mm_kernel(x_ref, w_ref, a_ref, h_ref, as_ref, ad_ref):
    xb = x_ref[...]
    h = jnp.dot(xb, w_ref[...], preferred_element_type=jnp.float32)
    h_ref[...] = h
    av = jnp.dot(h, a_ref[...], preferred_element_type=jnp.float32)
    as_ref[...] = _col_to_lanes(av, 0)
    ad_ref[...] = _col_to_lanes(av, 1)


def _build_tc_matmul(interpret=False):
    return pl.pallas_call(
        _mm_kernel,
        grid=(_GRID,),
        in_specs=[
            pl.BlockSpec((_BM, D), lambda i: (i, 0)),
            pl.BlockSpec((D, D), lambda i: (0, 0)),
            pl.BlockSpec((D, 128), lambda i: (0, 0)),
        ],
        out_specs=[
            pl.BlockSpec((_BM, D), lambda i: (i, 0)),
            pl.BlockSpec((_BM // 128, 128), lambda i: (i, 0)),
            pl.BlockSpec((_BM // 128, 128), lambda i: (i, 0)),
        ],
        out_shape=[
            jax.ShapeDtypeStruct((NPAD, D), jnp.float32),
            jax.ShapeDtypeStruct((NPAD // 128, 128), jnp.float32),
            jax.ShapeDtypeStruct((NPAD // 128, 128), jnp.float32),
        ],
        interpret=interpret,
    )


def _col_to_lanes(v, col):
    cc = lax.broadcasted_iota(jnp.int32, (_BM, 128), 1)
    vc = jnp.where(cc == col, v, 0.0)
    b = jnp.dot(vc, jnp.ones((128, 128), jnp.float32),
                preferred_element_type=jnp.float32)
    nc = lax.broadcasted_iota(jnp.int32, (_BM, 128), 0) % 128
    bm = jnp.where(nc == cc, b, 0.0)
    nq = lax.broadcasted_iota(jnp.int32, (_BM // 128, _BM), 0)
    kq = lax.broadcasted_iota(jnp.int32, (_BM // 128, _BM), 1) // 128
    r = jnp.where(nq == kq, 1.0, 0.0)
    return jnp.dot(r, bm, preferred_element_type=jnp.float32)


def _dn_rows(dn8):
    nq = lax.broadcasted_iota(jnp.int32, (_BM, _BM // 128), 0) // 128
    kq = lax.broadcasted_iota(jnp.int32, (_BM, _BM // 128), 1)
    r = jnp.where(nq == kq, 1.0, 0.0)
    t = jnp.dot(r, dn8, preferred_element_type=jnp.float32)
    nc = lax.broadcasted_iota(jnp.int32, (_BM, 128), 0) % 128
    cc = lax.broadcasted_iota(jnp.int32, (_BM, 128), 1)
    tm = jnp.where(nc == cc, t, 0.0)
    return jnp.dot(tm, jnp.ones((128, 128), jnp.float32),
                   preferred_element_type=jnp.float32)


def _epi_mm_kernel(p_ref, dn_ref, b_ref, w_ref, a_ref, h_ref, as_ref,
                   ad_ref):
    p = p_ref[0] + p_ref[1]
    dn = _dn_rows(dn_ref[0] + dn_ref[1] + 1e-16)
    y = p / dn + b_ref[...]
    y = jnp.maximum(y, 0.0)
    h = jnp.dot(y, w_ref[...], preferred_element_type=jnp.float32)
    h_ref[...] = h
    av = jnp.dot(h, a_ref[...], preferred_element_type=jnp.float32)
    as_ref[...] = _col_to_lanes(av, 0)
    ad_ref[...] = _col_to_lanes(av, 1)


def _build_tc_epi_matmul(interpret=False):
    return pl.pallas_call(
        _epi_mm_kernel,
        grid=(_GRID,),
        in_specs=[
            pl.BlockSpec((NC, _BM, D), lambda i: (0, i, 0)),
            pl.BlockSpec((NC, _BM // 128, 128), lambda i: (0, i, 0)),
            pl.BlockSpec((1, D), lambda i: (0, 0)),
            pl.BlockSpec((D, D), lambda i: (0, 0)),
            pl.BlockSpec((D, 128), lambda i: (0, 0)),
        ],
        out_specs=[
            pl.BlockSpec((_BM, D), lambda i: (i, 0)),
            pl.BlockSpec((_BM // 128, 128), lambda i: (i, 0)),
            pl.BlockSpec((_BM // 128, 128), lambda i: (i, 0)),
        ],
        out_shape=[
            jax.ShapeDtypeStruct((NPAD, D), jnp.float32),
            jax.ShapeDtypeStruct((NPAD // 128, 128), jnp.float32),
            jax.ShapeDtypeStruct((NPAD // 128, 128), jnp.float32),
        ],
        interpret=interpret,
    )


def _epi_kernel(p_ref, dn_ref, b_ref, o_ref):
    p = p_ref[0] + p_ref[1]
    dn = _dn_rows(dn_ref[0] + dn_ref[1] + 1e-16)
    o_ref[...] = p / dn + b_ref[...]


def _build_tc_epilogue(interpret=False):
    return pl.pallas_call(
        _epi_kernel,
        grid=(_GRID,),
        in_specs=[
            pl.BlockSpec((NC, _BM, D), lambda i: (0, i, 0)),
            pl.BlockSpec((NC, _BM // 128, 128), lambda i: (0, i, 0)),
            pl.BlockSpec((1, D), lambda i: (0, 0)),
        ],
        out_specs=pl.BlockSpec((_BM, D), lambda i: (i, 0)),
        out_shape=jax.ShapeDtypeStruct((NPAD, D), jnp.float32),
        interpret=interpret,
    )


_build_sc_edge = functools.lru_cache(maxsize=None)(_build_sc_edge)
_build_tc_matmul = functools.lru_cache(maxsize=None)(_build_tc_matmul)
_build_tc_epi_matmul = functools.lru_cache(maxsize=None)(_build_tc_epi_matmul)
_build_tc_epilogue = functools.lru_cache(maxsize=None)(_build_tc_epilogue)


@jax.jit
def kernel(x, edge_index, W1, a1_src, a1_dst, b1, W2, a2_src, a2_dst, b2):
    src = edge_index[0].astype(jnp.int32)
    dst = edge_index[1].astype(jnp.int32)
    pad_idx = N + (jnp.arange(EPAD - E, dtype=jnp.int32) % (NPAD - N))
    src2d = jnp.concatenate([src, pad_idx]).reshape(EDGE_ROWS, 128)
    dst2d = jnp.concatenate([dst, pad_idx]).reshape(EDGE_ROWS, 128)
    extra = jnp.broadcast_to(N + (jnp.arange(128, dtype=jnp.int32)
                                  % (NPAD - N)), (8, 128))
    src2d = jnp.concatenate([src2d, extra])
    dst2d = jnp.concatenate([dst2d, extra])

    xp = jnp.zeros((NPAD, D), jnp.float32).at[:N].set(x)
    A1 = (jnp.zeros((D, 128), jnp.float32)
          .at[:, 0].set(a1_src).at[:, 1].set(a1_dst))
    A2 = (jnp.zeros((D, 128), jnp.float32)
          .at[:, 0].set(a2_src).at[:, 1].set(a2_dst))

    sc_edge = _build_sc_edge()
    h1, as1, ad1 = _build_tc_matmul()(xp, W1, A1)
    outp1, dnp1 = sc_edge(h1, as1.reshape(NPAD), ad1.reshape(NPAD),
                          src2d, dst2d)
    h2, as2, ad2 = _build_tc_epi_matmul()(outp1, dnp1, b1.reshape(1, D),
                                          W2, A2)
    outp2, dnp2 = sc_edge(h2, as2.reshape(NPAD), ad2.reshape(NPAD),
                          src2d, dst2d)
    out = _build_tc_epilogue()(outp2, dnp2, b2.reshape(1, D))
    return out[:N]


# speedup vs baseline: 52.9566x; 1.0135x over previous
import functools

import jax
import jax.numpy as jnp
from jax import lax
from jax.experimental import pallas as pl
from jax.experimental.pallas import tpu as pltpu
from jax.experimental.pallas import tpu_sc as plsc

N = 10000
E = 320000
D = 128

NPAD = 10240
EPAD = 327680
NC = 2
NS = 16
GPT = EPAD // (NC * NS * 128)
EDGE_ROWS = EPAD // 128
RPT = NPAD // NS


NSB = GPT // 4


def _sc_edge_body(h_hbm, asrc_hbm, adst_hbm, src_hbm, dst_hbm,
                  outp_hbm, dnp_hbm,
                  src_sb, dst_sb, av_sb, bv_sb, w_sb, rows_v,
                  shared_out, shared_dn, shared_asrc, shared_adst,
                  sem_g0, sem_g1, sem_s0, sem_s1, sem_p, sem_d0, sem_d1):
    c = lax.axis_index("c")
    s = lax.axis_index("s")
    wid = c * NS + s
    sem_g = (sem_g0, sem_g1)
    sem_s = (sem_s0, sem_s1)
    sem_d = (sem_d0, sem_d1)

    @pl.when(s == 0)
    def _stage_alpha():
        pltpu.sync_copy(asrc_hbm, shared_asrc)
        pltpu.sync_copy(adst_hbm, shared_adst)

    zero16 = jnp.zeros((16,), jnp.float32)

    @pl.loop(0, 128)
    def _zero_rows(r):
        for m in range(8):
            rows_v[0, r, pl.ds(m * 16, 16)] = zero16

    for t in range(RPT // 128):
        pltpu.sync_copy(rows_v.at[0],
                        shared_out.at[pl.ds(s * RPT + t * 128, 128)])
        pltpu.sync_copy(rows_v.at[0, 0],
                        shared_dn.at[pl.ds(s * RPT + t * 128, 128)])
    plsc.subcore_barrier()


    def prep_sb(t1, pbn, drain=True):
        if drain:
            @pl.when(t1 >= 2)
            def _drain():
                for o in range(4):
                    pltpu.make_async_copy(w_sb.at[pbn, o],
                                          shared_dn.at[dst_sb.at[pbn, o]],
                                          sem_d[pbn]).wait()
        g0 = wid * GPT + 4 * t1
        d1 = pltpu.async_copy(src_hbm.at[pl.ds(g0, 4)], src_sb.at[pbn],
                              sem_p)
        d2 = pltpu.async_copy(dst_hbm.at[pl.ds(g0, 4)], dst_sb.at[pbn],
                              sem_p)
        d1.wait()
        d2.wait()
        ds = []
        for o in range(4):
            ds.append(pltpu.async_copy(shared_asrc.at[src_sb.at[pbn, o]],
                                       av_sb.at[o], sem_p))
            ds.append(pltpu.async_copy(shared_adst.at[dst_sb.at[pbn, o]],
                                       bv_sb.at[o], sem_p))
        for d in ds:
            d.wait()
        for o in range(4):
            for k in range(8):
                e = (av_sb[o, pl.ds(k * 16, 16)]
                     + bv_sb[o, pl.ds(k * 16, 16)])
                e = jnp.where(e > 0.0, e, 0.2 * e)
                w_sb[pbn, o, pl.ds(k * 16, 16)] = jnp.exp(e)

        @pl.when(t1 < NSB)
        def _dn():
            for o in range(4):
                pltpu.async_copy(w_sb.at[pbn, o],
                                 shared_dn.at[dst_sb.at[pbn, o]],
                                 sem_d[pbn], add=True)

    def start_gather(pb, o, rb):
        pltpu.async_copy(h_hbm.at[src_sb.at[pb, o]], rows_v.at[rb],
                         sem_g[rb])

    def wait_gather(pb, o, rb):
        pltpu.make_async_copy(h_hbm.at[src_sb.at[pb, o]], rows_v.at[rb],
                              sem_g[rb]).wait()

    def start_scatter(pb, o, rb):
        pltpu.async_copy(rows_v.at[rb], shared_out.at[dst_sb.at[pb, o]],
                         sem_s[rb], add=True)

    def wait_scatter(pb, o, rb):
        pltpu.make_async_copy(rows_v.at[rb],
                              shared_out.at[dst_sb.at[pb, o]],
                              sem_s[rb]).wait()

    def scale(rb, pb, o):
        @pl.loop(0, 8)
        def _scale(k):
            wvec = w_sb[pb, o, pl.ds(k * 16, 16)]
            for l in range(16):
                wb = lax.broadcast(wvec[l], (16,))
                r = k * 16 + l
                for m in range(8):
                    rows_v[rb, r, pl.ds(m * 16, 16)] = (
                        rows_v[rb, r, pl.ds(m * 16, 16)] * wb)

    def gbody(q, u):
        pb, o, rb = u // 4, u % 4, u % 2
        wait_gather(pb, o, rb)
        if o == 2:
            prep_sb(2 * q + u // 4 + 1, (u // 4 + 1) % 2)
        pu = (u - 1) % 8
        if u == 0:
            @pl.when(q > 0)
            def _ws():
                wait_scatter(pu // 4, pu % 4, pu % 2)
        else:
            wait_scatter(pu // 4, pu % 4, pu % 2)
        nu = (u + 1) % 8
        start_gather(nu // 4, nu % 4, nu % 2)
        scale(rb, pb, o)
        start_scatter(pb, o, rb)

    prep_sb(0, 0, drain=False)
    start_gather(0, 0, 0)

    @pl.loop(0, GPT // 8)
    def _edges(q):
        for u in range(8):
            gbody(q, u)

    wait_gather(0, 0, 0)
    wait_scatter(1, 3, 1)
    for o in range(4):
        pltpu.make_async_copy(w_sb.at[1, o],
                              shared_dn.at[dst_sb.at[1, o]],
                              sem_d[1]).wait()

    plsc.subcore_barrier()

    for t in range(RPT // 128):
        pltpu.sync_copy(shared_out.at[pl.ds(s * RPT + t * 128, 128)],
                        outp_hbm.at[c, pl.ds(s * RPT + t * 128, 128)])
        pltpu.sync_copy(shared_dn.at[pl.ds(s * RPT + t * 128, 128)],
                        dnp_hbm.at[c, s * (RPT // 128) + t])


def _build_sc_edge(interpret=False):
    mesh = plsc.VectorSubcoreMesh(core_axis_name="c", subcore_axis_name="s",
                                  num_cores=NC, num_subcores=NS)
    return pl.kernel(
        _sc_edge_body,
        out_type=[
            jax.ShapeDtypeStruct((NC, NPAD, D), jnp.float32),
            jax.ShapeDtypeStruct((NC, NPAD // 128, 128), jnp.float32),
        ],
        mesh=mesh,
        scratch_types=[
            pltpu.VMEM((2, 4, 128), jnp.int32),
            pltpu.VMEM((2, 4, 128), jnp.int32),
            pltpu.VMEM((4, 128), jnp.float32),
            pltpu.VMEM((4, 128), jnp.float32),
            pltpu.VMEM((2, 4, 128), jnp.float32),
            pltpu.VMEM((2, 128, D), jnp.float32),
            pltpu.VMEM_SHARED((NPAD, D), jnp.float32),
            pltpu.VMEM_SHARED((NPAD,), jnp.float32),
            pltpu.VMEM_SHARED((NPAD,), jnp.float32),
            pltpu.VMEM_SHARED((NPAD,), jnp.float32),
            pltpu.SemaphoreType.DMA,
            pltpu.SemaphoreType.DMA,
            pltpu.SemaphoreType.DMA,
            pltpu.SemaphoreType.DMA,
            pltpu.SemaphoreType.DMA,
            pltpu.SemaphoreType.DMA,
            pltpu.SemaphoreType.DMA,
        ],
        compiler_params=pltpu.CompilerParams(needs_layout_passes=False),
        interpret=interpret,
    )


_BM = 1024
_GRID = NPAD // _BM


def _mm_kernel(x_ref, w_ref, a_ref, h_ref, as_ref, ad_ref):
    i = pl.program_id(0)
    row = i * _BM + lax.broadcasted_iota(jnp.int32, (_BM, D), 0)
    xb = jnp.where(row < N, x_ref[...], 0.0)
    h = jnp.dot(xb, w_ref[...], preferred_element_type=jnp.float32)
    h_ref[...] = h
    av = jnp.dot(h, a_ref[...], preferred_element_type=jnp.float32)
    as_ref[...] = _col_to_lanes(av, 0)
    ad_ref[...] = _col_to_lanes(av, 1)


def _build_tc_matmul(interpret=False):
    return pl.pallas_call(
        _mm_kernel,
        grid=(_GRID,),
        in_specs=[
            pl.BlockSpec((_BM, D), lambda i: (i, 0)),
            pl.BlockSpec((D, D), lambda i: (0, 0)),
            pl.BlockSpec((D, 128), lambda i: (0, 0)),
        ],
        out_specs=[
            pl.BlockSpec((_BM, D), lambda i: (i, 0)),
            pl.BlockSpec((_BM // 128, 128), lambda i: (i, 0)),
            pl.BlockSpec((_BM // 128, 128), lambda i: (i, 0)),
        ],
        out_shape=[
            jax.ShapeDtypeStruct((NPAD, D), jnp.float32),
            jax.ShapeDtypeStruct((NPAD // 128, 128), jnp.float32),
            jax.ShapeDtypeStruct((NPAD // 128, 128), jnp.float32),
        ],
        interpret=interpret,
    )


def _col_to_lanes(v, col):
    cc = lax.broadcasted_iota(jnp.int32, (_BM, 128), 1)
    vc = jnp.where(cc == col, v, 0.0)
    b = jnp.dot(vc, jnp.ones((128, 128), jnp.float32),
                preferred_element_type=jnp.float32)
    nc = lax.broadcasted_iota(jnp.int32, (_BM, 128), 0) % 128
    bm = jnp.where(nc == cc, b, 0.0)
    nq = lax.broadcasted_iota(jnp.int32, (_BM // 128, _BM), 0)
    kq = lax.broadcasted_iota(jnp.int32, (_BM // 128, _BM), 1) // 128
    r = jnp.where(nq == kq, 1.0, 0.0)
    return jnp.dot(r, bm, preferred_element_type=jnp.float32)


def _dn_rows(dn8):
    nq = lax.broadcasted_iota(jnp.int32, (_BM, _BM // 128), 0) // 128
    kq = lax.broadcasted_iota(jnp.int32, (_BM, _BM // 128), 1)
    r = jnp.where(nq == kq, 1.0, 0.0)
    t = jnp.dot(r, dn8, preferred_element_type=jnp.float32)
    nc = lax.broadcasted_iota(jnp.int32, (_BM, 128), 0) % 128
    cc = lax.broadcasted_iota(jnp.int32, (_BM, 128), 1)
    tm = jnp.where(nc == cc, t, 0.0)
    return jnp.dot(tm, jnp.ones((128, 128), jnp.float32),
                   preferred_element_type=jnp.float32)


def _epi_mm_kernel(p_ref, dn_ref, b_ref, w_ref, a_ref, h_ref, as_ref,
                   ad_ref):
    p = p_ref[0] + p_ref[1]
    dn = _dn_rows(dn_ref[0] + dn_ref[1] + 1e-16)
    y = p / dn + b_ref[...]
    y = jnp.maximum(y, 0.0)
    h = jnp.dot(y, w_ref[...], preferred_element_type=jnp.float32)
    h_ref[...] = h
    av = jnp.dot(h, a_ref[...], preferred_element_type=jnp.float32)
    as_ref[...] = _col_to_lanes(av, 0)
    ad_ref[...] = _col_to_lanes(av, 1)


def _build_tc_epi_matmul(interpret=False):
    return pl.pallas_call(
        _epi_mm_kernel,
        grid=(_GRID,),
        in_specs=[
            pl.BlockSpec((NC, _BM, D), lambda i: (0, i, 0)),
            pl.BlockSpec((NC, _BM // 128, 128), lambda i: (0, i, 0)),
            pl.BlockSpec((1, D), lambda i: (0, 0)),
            pl.BlockSpec((D, D), lambda i: (0, 0)),
            pl.BlockSpec((D, 128), lambda i: (0, 0)),
        ],
        out_specs=[
            pl.BlockSpec((_BM, D), lambda i: (i, 0)),
            pl.BlockSpec((_BM // 128, 128), lambda i: (i, 0)),
            pl.BlockSpec((_BM // 128, 128), lambda i: (i, 0)),
        ],
        out_shape=[
            jax.ShapeDtypeStruct((NPAD, D), jnp.float32),
            jax.ShapeDtypeStruct((NPAD // 128, 128), jnp.float32),
            jax.ShapeDtypeStruct((NPAD // 128, 128), jnp.float32),
        ],
        interpret=interpret,
    )


def _epi_kernel(p_ref, dn_ref, b_ref, o_ref):
    p = p_ref[0] + p_ref[1]
    dn = _dn_rows(dn_ref[0] + dn_ref[1] + 1e-16)
    o_ref[...] = p / dn + b_ref[...]


def _build_tc_epilogue(interpret=False):
    return pl.pallas_call(
        _epi_kernel,
        grid=(_GRID,),
        in_specs=[
            pl.BlockSpec((NC, _BM, D), lambda i: (0, i, 0)),
            pl.BlockSpec((NC, _BM // 128, 128), lambda i: (0, i, 0)),
            pl.BlockSpec((1, D), lambda i: (0, 0)),
        ],
        out_specs=pl.BlockSpec((_BM, D), lambda i: (i, 0)),
        out_shape=jax.ShapeDtypeStruct((N, D), jnp.float32),
        interpret=interpret,
    )


_build_sc_edge = functools.lru_cache(maxsize=None)(_build_sc_edge)
_build_tc_matmul = functools.lru_cache(maxsize=None)(_build_tc_matmul)
_build_tc_epi_matmul = functools.lru_cache(maxsize=None)(_build_tc_epi_matmul)
_build_tc_epilogue = functools.lru_cache(maxsize=None)(_build_tc_epilogue)


@jax.jit
def kernel(x, edge_index, W1, a1_src, a1_dst, b1, W2, a2_src, a2_dst, b2):
    src = edge_index[0].astype(jnp.int32)
    dst = edge_index[1].astype(jnp.int32)
    pad_idx = N + (jnp.arange(EPAD - E, dtype=jnp.int32) % (NPAD - N))
    src2d = jnp.concatenate([src, pad_idx]).reshape(EDGE_ROWS, 128)
    dst2d = jnp.concatenate([dst, pad_idx]).reshape(EDGE_ROWS, 128)
    extra = jnp.broadcast_to(N + (jnp.arange(128, dtype=jnp.int32)
                                  % (NPAD - N)), (8, 128))
    src2d = jnp.concatenate([src2d, extra])
    dst2d = jnp.concatenate([dst2d, extra])

    A1 = (jnp.zeros((D, 128), jnp.float32)
          .at[:, 0].set(a1_src).at[:, 1].set(a1_dst))
    A2 = (jnp.zeros((D, 128), jnp.float32)
          .at[:, 0].set(a2_src).at[:, 1].set(a2_dst))

    sc_edge = _build_sc_edge()
    h1, as1, ad1 = _build_tc_matmul()(x, W1, A1)
    outp1, dnp1 = sc_edge(h1, as1.reshape(NPAD), ad1.reshape(NPAD),
                          src2d, dst2d)
    h2, as2, ad2 = _build_tc_epi_matmul()(outp1, dnp1, b1.reshape(1, D),
                                          W2, A2)
    outp2, dnp2 = sc_edge(h2, as2.reshape(NPAD), ad2.reshape(NPAD),
                          src2d, dst2d)
    return _build_tc_epilogue()(outp2, dnp2, b2.reshape(1, D))


# speedup vs baseline: 54.7496x; 1.0339x over previous
import functools

import jax
import jax.numpy as jnp
from jax import lax
from jax.experimental import pallas as pl
from jax.experimental.pallas import tpu as pltpu
from jax.experimental.pallas import tpu_sc as plsc

N = 10000
E = 320000
D = 128

NPAD = 10240
EPAD = 327680
NC = 2
NS = 16
GPT = EPAD // (NC * NS * 128)
EDGE_ROWS = EPAD // 128
RPT = NPAD // NS


NSB = GPT // 4


def _sc_edge_body(h_hbm, asrc_hbm, adst_hbm, src_hbm, dst_hbm,
                  outp_hbm, dnp_hbm,
                  src_sb, dst_sb, av_sb, bv_sb, w_sb, rows_v,
                  shared_out, shared_dn, shared_asrc, shared_adst,
                  sem_g0, sem_g1, sem_s0, sem_s1, sem_p, sem_d0, sem_d1):
    c = lax.axis_index("c")
    s = lax.axis_index("s")
    wid = c * NS + s
    sem_g = (sem_g0, sem_g1)
    sem_s = (sem_s0, sem_s1)
    sem_d = (sem_d0, sem_d1)

    @pl.when(s == 0)
    def _stage_alpha():
        pltpu.sync_copy(asrc_hbm, shared_asrc)
        pltpu.sync_copy(adst_hbm, shared_adst)

    zero16 = jnp.zeros((16,), jnp.float32)

    @pl.loop(0, 128)
    def _zero_rows(r):
        for m in range(8):
            rows_v[0, r, pl.ds(m * 16, 16)] = zero16

    for t in range(RPT // 128):
        pltpu.sync_copy(rows_v.at[0],
                        shared_out.at[pl.ds(s * RPT + t * 128, 128)])
        pltpu.sync_copy(rows_v.at[0, 0],
                        shared_dn.at[pl.ds(s * RPT + t * 128, 128)])
    plsc.subcore_barrier()


    def fire_idx(t1, pbn):
        g0 = wid * GPT + 4 * t1
        pltpu.async_copy(src_hbm.at[pl.ds(g0, 4)], src_sb.at[pbn], sem_p)
        pltpu.async_copy(dst_hbm.at[pl.ds(g0, 4)], dst_sb.at[pbn], sem_p)

    def drain_dn(pbn):
        for o in range(4):
            pltpu.make_async_copy(w_sb.at[pbn, o],
                                  shared_dn.at[dst_sb.at[pbn, o]],
                                  sem_d[pbn]).wait()

    def prep_sb(t1, pbn):
        g0 = wid * GPT + 4 * t1
        pltpu.make_async_copy(src_hbm.at[pl.ds(g0, 4)], src_sb.at[pbn],
                              sem_p).wait()
        pltpu.make_async_copy(dst_hbm.at[pl.ds(g0, 4)], dst_sb.at[pbn],
                              sem_p).wait()
        ds = []
        for o in range(4):
            ds.append(pltpu.async_copy(shared_asrc.at[src_sb.at[pbn, o]],
                                       av_sb.at[o], sem_p))
            ds.append(pltpu.async_copy(shared_adst.at[dst_sb.at[pbn, o]],
                                       bv_sb.at[o], sem_p))
        for d in ds:
            d.wait()
        for o in range(4):
            for k in range(8):
                e = (av_sb[o, pl.ds(k * 16, 16)]
                     + bv_sb[o, pl.ds(k * 16, 16)])
                e = jnp.where(e > 0.0, e, 0.2 * e)
                w_sb[pbn, o, pl.ds(k * 16, 16)] = jnp.exp(e)

        @pl.when(t1 < NSB)
        def _dn():
            for o in range(4):
                pltpu.async_copy(w_sb.at[pbn, o],
                                 shared_dn.at[dst_sb.at[pbn, o]],
                                 sem_d[pbn], add=True)

    def start_gather(pb, o, rb):
        pltpu.async_copy(h_hbm.at[src_sb.at[pb, o]], rows_v.at[rb],
                         sem_g[rb])

    def wait_gather(pb, o, rb):
        pltpu.make_async_copy(h_hbm.at[src_sb.at[pb, o]], rows_v.at[rb],
                              sem_g[rb]).wait()

    def start_scatter(pb, o, rb):
        pltpu.async_copy(rows_v.at[rb], shared_out.at[dst_sb.at[pb, o]],
                         sem_s[rb], add=True)

    def wait_scatter(pb, o, rb):
        pltpu.make_async_copy(rows_v.at[rb],
                              shared_out.at[dst_sb.at[pb, o]],
                              sem_s[rb]).wait()

    def scale(rb, pb, o):
        @pl.loop(0, 8)
        def _scale(k):
            wvec = w_sb[pb, o, pl.ds(k * 16, 16)]
            for l in range(16):
                wb = lax.broadcast(wvec[l], (16,))
                r = k * 16 + l
                for m in range(8):
                    rows_v[rb, r, pl.ds(m * 16, 16)] = (
                        rows_v[rb, r, pl.ds(m * 16, 16)] * wb)

    def gbody(q, u):
        pb, o, rb = u // 4, u % 4, u % 2
        wait_gather(pb, o, rb)
        if o == 2:
            prep_sb(2 * q + u // 4 + 1, (u // 4 + 1) % 2)
        pu = (u - 1) % 8
        if u == 0:
            @pl.when(q > 0)
            def _ws():
                wait_scatter(pu // 4, pu % 4, pu % 2)
        else:
            wait_scatter(pu // 4, pu % 4, pu % 2)
        if o == 0:
            nb = (u // 4 + 1) % 2
            if u == 0:
                @pl.when(q > 0)
                def _dd():
                    drain_dn(nb)
            else:
                drain_dn(nb)
            fire_idx(2 * q + u // 4 + 1, nb)
        nu = (u + 1) % 8
        start_gather(nu // 4, nu % 4, nu % 2)
        scale(rb, pb, o)
        start_scatter(pb, o, rb)

    fire_idx(0, 0)
    prep_sb(0, 0)
    start_gather(0, 0, 0)

    @pl.loop(0, GPT // 8)
    def _edges(q):
        for u in range(8):
            gbody(q, u)

    wait_gather(0, 0, 0)
    wait_scatter(1, 3, 1)
    for o in range(4):
        pltpu.make_async_copy(w_sb.at[1, o],
                              shared_dn.at[dst_sb.at[1, o]],
                              sem_d[1]).wait()

    plsc.subcore_barrier()

    for t in range(RPT // 128):
        pltpu.sync_copy(shared_out.at[pl.ds(s * RPT + t * 128, 128)],
                        outp_hbm.at[c, pl.ds(s * RPT + t * 128, 128)])
        pltpu.sync_copy(shared_dn.at[pl.ds(s * RPT + t * 128, 128)],
                        dnp_hbm.at[c, s * (RPT // 128) + t])


def _build_sc_edge(interpret=False):
    mesh = plsc.VectorSubcoreMesh(core_axis_name="c", subcore_axis_name="s",
                                  num_cores=NC, num_subcores=NS)
    return pl.kernel(
        _sc_edge_body,
        out_type=[
            jax.ShapeDtypeStruct((NC, NPAD, D), jnp.float32),
            jax.ShapeDtypeStruct((NC, NPAD // 128, 128), jnp.float32),
        ],
        mesh=mesh,
        scratch_types=[
            pltpu.VMEM((2, 4, 128), jnp.int32),
            pltpu.VMEM((2, 4, 128), jnp.int32),
            pltpu.VMEM((4, 128), jnp.float32),
            pltpu.VMEM((4, 128), jnp.float32),
            pltpu.VMEM((2, 4, 128), jnp.float32),
            pltpu.VMEM((2, 128, D), jnp.float32),
            pltpu.VMEM_SHARED((NPAD, D), jnp.float32),
            pltpu.VMEM_SHARED((NPAD,), jnp.float32),
            pltpu.VMEM_SHARED((NPAD,), jnp.float32),
            pltpu.VMEM_SHARED((NPAD,), jnp.float32),
            pltpu.SemaphoreType.DMA,
            pltpu.SemaphoreType.DMA,
            pltpu.SemaphoreType.DMA,
            pltpu.SemaphoreType.DMA,
            pltpu.SemaphoreType.DMA,
            pltpu.SemaphoreType.DMA,
            pltpu.SemaphoreType.DMA,
        ],
        compiler_params=pltpu.CompilerParams(needs_layout_passes=False),
        interpret=interpret,
    )


_BM = 1024
_GRID = NPAD // _BM


def _mm_kernel(x_ref, w_ref, a_ref, h_ref, as_ref, ad_ref):
    i = pl.program_id(0)
    row = i * _BM + lax.broadcasted_iota(jnp.int32, (_BM, D), 0)
    xb = jnp.where(row < N, x_ref[...], 0.0)
    h = jnp.dot(xb, w_ref[...], preferred_element_type=jnp.float32)
    h_ref[...] = h
    av = jnp.dot(h, a_ref[...], preferred_element_type=jnp.float32)
    as_ref[...] = _col_to_lanes(av, 0)
    ad_ref[...] = _col_to_lanes(av, 1)


def _build_tc_matmul(interpret=False):
    return pl.pallas_call(
        _mm_kernel,
        grid=(_GRID,),
        in_specs=[
            pl.BlockSpec((_BM, D), lambda i: (i, 0)),
            pl.BlockSpec((D, D), lambda i: (0, 0)),
            pl.BlockSpec((D, 128), lambda i: (0, 0)),
        ],
        out_specs=[
            pl.BlockSpec((_BM, D), lambda i: (i, 0)),
            pl.BlockSpec((_BM // 128, 128), lambda i: (i, 0)),
            pl.BlockSpec((_BM // 128, 128), lambda i: (i, 0)),
        ],
        out_shape=[
            jax.ShapeDtypeStruct((NPAD, D), jnp.float32),
            jax.ShapeDtypeStruct((NPAD // 128, 128), jnp.float32),
            jax.ShapeDtypeStruct((NPAD // 128, 128), jnp.float32),
        ],
        interpret=interpret,
    )


def _col_to_lanes(v, col):
    cc = lax.broadcasted_iota(jnp.int32, (_BM, 128), 1)
    vc = jnp.where(cc == col, v, 0.0)
    b = jnp.dot(vc, jnp.ones((128, 128), jnp.float32),
                preferred_element_type=jnp.float32)
    nc = lax.broadcasted_iota(jnp.int32, (_BM, 128), 0) % 128
    bm = jnp.where(nc == cc, b, 0.0)
    nq = lax.broadcasted_iota(jnp.int32, (_BM // 128, _BM), 0)
    kq = lax.broadcasted_iota(jnp.int32, (_BM // 128, _BM), 1) // 128
    r = jnp.where(nq == kq, 1.0, 0.0)
    return jnp.dot(r, bm, preferred_element_type=jnp.float32)


def _dn_rows(dn8):
    nq = lax.broadcasted_iota(jnp.int32, (_BM, _BM // 128), 0) // 128
    kq = lax.broadcasted_iota(jnp.int32, (_BM, _BM // 128), 1)
    r = jnp.where(nq == kq, 1.0, 0.0)
    t = jnp.dot(r, dn8, preferred_element_type=jnp.float32)
    nc = lax.broadcasted_iota(jnp.int32, (_BM, 128), 0) % 128
    cc = lax.broadcasted_iota(jnp.int32, (_BM, 128), 1)
    tm = jnp.where(nc == cc, t, 0.0)
    return jnp.dot(tm, jnp.ones((128, 128), jnp.float32),
                   preferred_element_type=jnp.float32)


def _epi_mm_kernel(p_ref, dn_ref, b_ref, w_ref, a_ref, h_ref, as_ref,
                   ad_ref):
    p = p_ref[0] + p_ref[1]
    dn = _dn_rows(dn_ref[0] + dn_ref[1] + 1e-16)
    y = p / dn + b_ref[...]
    y = jnp.maximum(y, 0.0)
    h = jnp.dot(y, w_ref[...], preferred_element_type=jnp.float32)
    h_ref[...] = h
    av = jnp.dot(h, a_ref[...], preferred_element_type=jnp.float32)
    as_ref[...] = _col_to_lanes(av, 0)
    ad_ref[...] = _col_to_lanes(av, 1)


def _build_tc_epi_matmul(interpret=False):
    return pl.pallas_call(
        _epi_mm_kernel,
        grid=(_GRID,),
        in_specs=[
            pl.BlockSpec((NC, _BM, D), lambda i: (0, i, 0)),
            pl.BlockSpec((NC, _BM // 128, 128), lambda i: (0, i, 0)),
            pl.BlockSpec((1, D), lambda i: (0, 0)),
            pl.BlockSpec((D, D), lambda i: (0, 0)),
            pl.BlockSpec((D, 128), lambda i: (0, 0)),
        ],
        out_specs=[
            pl.BlockSpec((_BM, D), lambda i: (i, 0)),
            pl.BlockSpec((_BM // 128, 128), lambda i: (i, 0)),
            pl.BlockSpec((_BM // 128, 128), lambda i: (i, 0)),
        ],
        out_shape=[
            jax.ShapeDtypeStruct((NPAD, D), jnp.float32),
            jax.ShapeDtypeStruct((NPAD // 128, 128), jnp.float32),
            jax.ShapeDtypeStruct((NPAD // 128, 128), jnp.float32),
        ],
        interpret=interpret,
    )


def _epi_kernel(p_ref, dn_ref, b_ref, o_ref):
    p = p_ref[0] + p_ref[1]
    dn = _dn_rows(dn_ref[0] + dn_ref[1] + 1e-16)
    o_ref[...] = p / dn + b_ref[...]


def _build_tc_epilogue(interpret=False):
    return pl.pallas_call(
        _epi_kernel,
        grid=(_GRID,),
        in_specs=[
            pl.BlockSpec((NC, _BM, D), lambda i: (0, i, 0)),
            pl.BlockSpec((NC, _BM // 128, 128), lambda i: (0, i, 0)),
            pl.BlockSpec((1, D), lambda i: (0, 0)),
        ],
        out_specs=pl.BlockSpec((_BM, D), lambda i: (i, 0)),
        out_shape=jax.ShapeDtypeStruct((N, D), jnp.float32),
        interpret=interpret,
    )


_build_sc_edge = functools.lru_cache(maxsize=None)(_build_sc_edge)
_build_tc_matmul = functools.lru_cache(maxsize=None)(_build_tc_matmul)
_build_tc_epi_matmul = functools.lru_cache(maxsize=None)(_build_tc_epi_matmul)
_build_tc_epilogue = functools.lru_cache(maxsize=None)(_build_tc_epilogue)


@jax.jit
def kernel(x, edge_index, W1, a1_src, a1_dst, b1, W2, a2_src, a2_dst, b2):
    src = edge_index[0].astype(jnp.int32)
    dst = edge_index[1].astype(jnp.int32)
    pad_idx = N + (jnp.arange(EPAD - E, dtype=jnp.int32) % (NPAD - N))
    src2d = jnp.concatenate([src, pad_idx]).reshape(EDGE_ROWS, 128)
    dst2d = jnp.concatenate([dst, pad_idx]).reshape(EDGE_ROWS, 128)
    extra = jnp.broadcast_to(N + (jnp.arange(128, dtype=jnp.int32)
                                  % (NPAD - N)), (8, 128))
    src2d = jnp.concatenate([src2d, extra])
    dst2d = jnp.concatenate([dst2d, extra])

    A1 = (jnp.zeros((D, 128), jnp.float32)
          .at[:, 0].set(a1_src).at[:, 1].set(a1_dst))
    A2 = (jnp.zeros((D, 128), jnp.float32)
          .at[:, 0].set(a2_src).at[:, 1].set(a2_dst))

    sc_edge = _build_sc_edge()
    h1, as1, ad1 = _build_tc_matmul()(x, W1, A1)
    outp1, dnp1 = sc_edge(h1, as1.reshape(NPAD), ad1.reshape(NPAD),
                          src2d, dst2d)
    h2, as2, ad2 = _build_tc_epi_matmul()(outp1, dnp1, b1.reshape(1, D),
                                          W2, A2)
    outp2, dnp2 = sc_edge(h2, as2.reshape(NPAD), ad2.reshape(NPAD),
                          src2d, dst2d)
    return _build_tc_epilogue()(outp2, dnp2, b2.reshape(1, D))
